# baseline (speedup 1.0000x reference)
import dataclasses
import functools
import math

import jax
import jax.numpy as jnp
from jax import lax
from jax.experimental import pallas as pl
from jax.experimental.pallas import tpu as pltpu
from jax.experimental.pallas import tpu_sc as plsc

N_NODE = 10000
RW = 128
NH = 5
CHUNK = 64
ROWS_T = 632


def _cdiv(a, b):
    return (a + b - 1) // b


_T_GATHER = True
_T_SCALE = True
_T_SCAT = True
_T_DEN = True


def _sc_params():
    cp = pltpu.CompilerParams()
    if "needs_layout_passes" in pltpu.CompilerParams.__dataclass_fields__:
        cp = dataclasses.replace(cp, needs_layout_passes=False)
    return cp


def _worker_range(wid, GT, chunk):
    U = _cdiv(GT, chunk)
    ub, ur = divmod(U, 32)
    u0 = wid * ub + jnp.minimum(wid, ur)
    nu = ub + jnp.where(wid < ur, 1, 0)
    g0 = u0 * chunk
    ng = jnp.minimum(nu * chunk, jnp.maximum(GT - g0, 0))
    max_chunks = ub + 1
    return g0, ng, max_chunks


CHUNK_A = 32


def _make_sc_logits(E, has_prev, is_dst):
    GT = E // 16
    mesh = plsc.VectorSubcoreMesh(core_axis_name="c", subcore_axis_name="s")

    scratch = dict(
        tab_v=pltpu.VMEM((N_NODE * 5,), jnp.float32),
        idx_c=pltpu.VMEM((CHUNK_A, 16), jnp.int32),
        s_st=pltpu.VMEM((CHUNK_A * 128,), jnp.float32),
    )
    if has_prev:
        scratch['pv_c'] = pltpu.VMEM((CHUNK_A * 128,), jnp.float32)

    def body(*refs):
        names = ['tab'] + (['prev'] if has_prev else []) + ['idxg', 's_o']
        names += list(scratch.keys())
        r = dict(zip(names, refs))

        cid = lax.axis_index("c")
        sid = lax.axis_index("s")
        wid = sid * 2 + cid
        iota = lax.iota(jnp.int32, 16)
        zv = jnp.zeros((16,), jnp.float32)

        pltpu.sync_copy(r['tab'], r['tab_v'])

        g0, ng, max_chunks = _worker_range(wid, GT, CHUNK_A)

        @pl.loop(0, max_chunks)
        def _(ci):
            @pl.when(ci * CHUNK_A < ng)
            def _():
                gbase = g0 + ci * CHUNK_A
                glen = jnp.minimum(CHUNK_A, ng - ci * CHUNK_A)
                pltpu.sync_copy(r['idxg'].at[pl.ds(gbase, CHUNK_A), :],
                                r['idx_c'])
                if has_prev:
                    pltpu.sync_copy(
                        r['prev'].at[pl.ds(gbase * 128, CHUNK_A * 128)],
                        r['pv_c'])

                @pl.loop(0, CHUNK_A)
                def _(gi):
                    nodev = r['idx_c'][gi]
                    for h in range(NH):
                        a = plsc.load_gather(r['tab_v'], [nodev * 5 + h])
                        if has_prev:
                            a = a + plsc.load_gather(
                                r['pv_c'], [gi * 128 + iota * 8 + h])
                        if is_dst:
                            a = jnp.maximum(a, 0.2 * a)
                            a = jnp.exp(a)
                        plsc.store_scatter(
                            r['s_st'], [gi * 128 + iota * 8 + h], a)
                    for c in range(5, 8):
                        plsc.store_scatter(
                            r['s_st'], [gi * 128 + iota * 8 + c], zv)
                pltpu.sync_copy(
                    r['s_st'],
                    r['s_o'].at[pl.ds(gbase * 128, CHUNK_A * 128)])

    return pl.kernel(
        body,
        out_type=jax.ShapeDtypeStruct(((GT + CHUNK) * 128,), jnp.float32),
        mesh=mesh, compiler_params=_sc_params(),
        scratch_types=list(scratch.values()))


CHUNK_B = 32
MGG = 4


def _make_sc_pay(E, C, col_off, PW, lo):
    GT = E // 16
    SPLIT = 8128
    SH_ROWS = 8192 if lo else 2048
    REAL = SPLIT if lo else N_NODE - SPLIT
    RPT = 512 if lo else 120
    NMG = CHUNK_B // MGG
    mesh = plsc.VectorSubcoreMesh(core_axis_name="c", subcore_axis_name="s")

    scratch = dict(
        src_f=pltpu.VMEM((CHUNK_B * 16,), jnp.int32),
        dst_f=pltpu.VMEM((CHUNK_B * 16,), jnp.int32),
        dst_c2=pltpu.VMEM((NMG, 64), jnp.int32),
        s_c=pltpu.VMEM((CHUNK_B * 128,), jnp.float32),
        rows0=pltpu.VMEM((64, 128), jnp.float32),
        rows1=pltpu.VMEM((64, 128), jnp.float32),
        gsem0=pltpu.SemaphoreType.DMA,
        gsem1=pltpu.SemaphoreType.DMA,
        ssem0=pltpu.SemaphoreType.DMA,
        ssem1=pltpu.SemaphoreType.DMA,
        acc_sh=pltpu.VMEM_SHARED((SH_ROWS, 128), jnp.float32),
    )

    def body(hs, s8, src_h, dst_h, acc_o, src_f, dst_f, dst_c2, s_c,
             rows0, rows1, gsem0, gsem1, ssem0, ssem1, acc_sh):
        cid = lax.axis_index("c")
        sid = lax.axis_index("s")
        wid = sid * 2 + cid
        iota = lax.iota(jnp.int32, 16)
        zv = jnp.zeros((16,), jnp.float32)
        rows = [rows0, rows1]
        gsem = [gsem0, gsem1]
        ssem = [ssem0, ssem1]

        @pl.loop(0, 64)
        def _(i):
            for k in range(8):
                rows0[i, pl.ds(16 * k, 16)] = zv

        z0 = sid * (SH_ROWS // 16)
        nzc = _cdiv(SH_ROWS // 16, 64)

        @pl.loop(0, nzc)
        def _(czi):
            start = jnp.minimum(z0 + czi * 64, z0 + SH_ROWS // 16 - 64)
            pltpu.sync_copy(rows0, acc_sh.at[pl.ds(start, 64), :])

        plsc.subcore_barrier()

        g0, ng, max_chunks = _worker_range(wid, GT, CHUNK_B)

        hvecs = []
        for k in range(8):
            gcol = iota + (16 * k + col_off)
            hv = jnp.where(
                gcol < PW, gcol // C,
                jnp.where(gcol < PW + 5, gcol - PW, 7))
            hvecs.append(hv)

        @pl.loop(0, max_chunks)
        def _(ci):
            @pl.when(ci * CHUNK_B < ng)
            def _():
                gbase = g0 + ci * CHUNK_B
                glen = jnp.minimum(CHUNK_B, ng - ci * CHUNK_B)
                pltpu.sync_copy(src_h.at[pl.ds(gbase * 16, CHUNK_B * 16)],
                                src_f)
                pltpu.sync_copy(dst_h.at[pl.ds(gbase * 16, CHUNK_B * 16)],
                                dst_f)
                pltpu.sync_copy(s8.at[pl.ds(gbase * 128, CHUNK_B * 128)], s_c)

                @pl.loop(0, CHUNK_B)
                def _(i):
                    dstv = dst_f[pl.ds(i * 16, 16)]
                    if lo:
                        dstv = jnp.where(dstv < SPLIT, dstv, SPLIT + sid)
                    else:
                        dstv = jnp.where(dstv >= SPLIT, dstv - SPLIT,
                                         REAL + sid)
                    dstv = jnp.where(i < glen, dstv,
                                     (SPLIT if lo else REAL) + sid)
                    dst_c2[i // MGG, pl.ds((i % MGG) * 16, 16)] = dstv

                def scale(buf, mg):
                    @pl.loop(0, 64)
                    def _(rr):
                        sbase = (mg * 64 + rr) * 8
                        for k in range(8):
                            sc = plsc.load_gather(s_c, [sbase + hvecs[k]])
                            buf[rr, pl.ds(16 * k, 16)] = (
                                buf[rr, pl.ds(16 * k, 16)] * sc)

                def gath(mg, bi):
                    return pltpu.async_copy(
                        hs.at[src_f.at[pl.ds(mg * 64, 64)]], rows[bi],
                        gsem[bi])

                def scat(mg, bi):
                    return pltpu.async_copy(
                        rows[bi], acc_sh.at[dst_c2.at[mg]], ssem[bi],
                        add=True)

                pend_g = {0: gath(0, 0), 1: gath(1, 1)}
                pend_s = {}
                for mg in range(NMG):
                    bi = mg & 1
                    pend_g[mg].wait()
                    scale(rows[bi], mg)
                    if mg >= 1:
                        bp = 1 - bi
                        pend_s[mg - 1].wait()
                        if mg + 1 < NMG:
                            pend_g[mg + 1] = gath(mg + 1, bp)
                    pend_s[mg] = scat(mg, bi)
                pend_s[NMG - 1].wait()

        plsc.subcore_barrier()

        r0 = sid * RPT
        rtop = jnp.minimum(r0 + RPT, REAL) - 64
        noc = _cdiv(RPT, 64)

        @pl.loop(0, noc)
        def _(co):
            start = jnp.minimum(r0 + co * 64, rtop)
            pltpu.sync_copy(acc_sh.at[pl.ds(start, 64), :], rows0)
            pltpu.sync_copy(rows0, acc_o.at[cid, pl.ds(start, 64), :])

    return pl.kernel(
        body,
        out_type=jax.ShapeDtypeStruct((2, REAL, 128), jnp.float32),
        mesh=mesh, compiler_params=_sc_params(),
        scratch_types=list(scratch.values()))


def _make_sc_seg(E):
    GT = E // 16
    U = _cdiv(GT, 8)
    ub, ur = divmod(U, 32)
    max_chunks = _cdiv((ub + 1) * 8, CHUNK)
    NP = 10240

    mesh = plsc.VectorSubcoreMesh(core_axis_name="c", subcore_axis_name="s")

    def body(t_hbm, srcg, dstg, rel_o, t_v, src_c, dst_c, part_v):
        cid = lax.axis_index("c")
        sid = lax.axis_index("s")
        wid = sid * 2 + cid
        zv = jnp.zeros((16,), jnp.float32)

        @pl.loop(0, NP // 16)
        def _(i):
            part_v[pl.ds(i * 16, 16)] = zv

        pltpu.sync_copy(t_hbm, t_v)

        u0 = wid * ub + jnp.minimum(wid, ur)
        nu = ub + jnp.where(wid < ur, 1, 0)
        g0 = u0 * 8
        ng = jnp.minimum(nu * 8, GT - g0)

        @pl.loop(0, max_chunks)
        def _(ci):
            @pl.when(ci * CHUNK < ng)
            def _():
                gbase = g0 + ci * CHUNK
                glen = jnp.minimum(CHUNK, ng - ci * CHUNK)
                pltpu.sync_copy(srcg.at[pl.ds(gbase, CHUNK), :], src_c)
                pltpu.sync_copy(dstg.at[pl.ds(gbase, CHUNK), :], dst_c)

                @pl.loop(0, CHUNK)
                def _(gi):
                    @pl.when(gi < glen)
                    def _():
                        srcv = src_c[gi]
                        dstv = dst_c[gi]
                        t16 = plsc.load_gather(t_v, [srcv])
                        plsc.addupdate_scatter(part_v, [dstv], t16)

        pltpu.sync_copy(part_v, rel_o.at[pl.ds(wid * NP, NP)])

    return pl.kernel(
        body,
        out_type=jax.ShapeDtypeStruct((32 * NP,), jnp.float32),
        mesh=mesh,
        compiler_params=_sc_params(),
        scratch_types=[
            pltpu.VMEM((N_NODE,), jnp.float32),
            pltpu.VMEM((CHUNK, 16), jnp.int32),
            pltpu.VMEM((CHUNK, 16), jnp.int32),
            pltpu.VMEM((NP,), jnp.float32),
        ])


MMB = 400


def _mm(x, w):
    N, D = x.shape
    K = w.shape[1]

    def body(x_ref, w_ref, o_ref):
        o_ref[...] = jnp.dot(x_ref[...], w_ref[...],
                             preferred_element_type=jnp.float32)

    return pl.pallas_call(
        body,
        grid=(N // MMB,),
        in_specs=[pl.BlockSpec((MMB, D), lambda i: (i, 0)),
                  pl.BlockSpec((D, K), lambda i: (0, 0))],
        out_specs=pl.BlockSpec((MMB, K), lambda i: (i, 0)),
        out_shape=jax.ShapeDtypeStruct((N, K), jnp.float32),
    )(x, w)


def _finalize_concat(accA, accB, biasA, biasB):
    N = accA.shape[1]

    def body(aA, aB, bA, bB, o_ref):
        def half(a_ref, b_ref):
            a = a_ref[0] + a_ref[1]
            d = a[:, 100:105]
            dexp = jnp.broadcast_to(d[:, :, None], (MMB, NH, 100 // NH))
            dexp = dexp.reshape(MMB, 100)
            return a[:, :100] / (dexp + 1e-16) + b_ref[0, :100]
        o_ref[...] = jnp.concatenate([half(aA, bA), half(aB, bB)], axis=1)

    bs = pl.BlockSpec((2, MMB, 128), lambda i: (0, i, 0))
    return pl.pallas_call(
        body,
        grid=(N // MMB,),
        in_specs=[bs, bs,
                  pl.BlockSpec((1, 128), lambda i: (0, 0)),
                  pl.BlockSpec((1, 128), lambda i: (0, 0))],
        out_specs=pl.BlockSpec((MMB, 200), lambda i: (i, 0)),
        out_shape=jax.ShapeDtypeStruct((N, 200), jnp.float32),
    )(accA, accB, biasA, biasB)


def _gln_stats(x, batch2d):
    N, C = x.shape
    NB = N // MMB

    def body(x_ref, b_ref, o_ref, s_ref):
        i = pl.program_id(0)

        @pl.when(i == 0)
        def _():
            s_ref[...] = jnp.zeros_like(s_ref)

        xb = x_ref[...]
        onehot = (b_ref[...] == lax.broadcasted_iota(
            jnp.int32, (MMB, 128), 1)).astype(jnp.float32)
        s_ref[0, :] += (onehot * xb.sum(axis=1, keepdims=True)).sum(axis=0)
        s_ref[1, :] += (onehot * (xb * xb).sum(axis=1, keepdims=True)).sum(axis=0)
        s_ref[2, :] += onehot.sum(axis=0)

        cnt = jnp.maximum(s_ref[2, :] * C, 1.0)
        mean = s_ref[0, :] / cnt
        var = s_ref[1, :] / cnt - mean * mean
        o_ref[0, :] = mean
        o_ref[1, :] = lax.rsqrt(var + 1e-5)

    return pl.pallas_call(
        body,
        grid=(NB,),
        in_specs=[pl.BlockSpec((MMB, C), lambda i: (i, 0)),
                  pl.BlockSpec((MMB, 1), lambda i: (i, 0))],
        out_specs=pl.BlockSpec((2, 128), lambda i: (0, 0)),
        out_shape=jax.ShapeDtypeStruct((2, 128), jnp.float32),
        scratch_shapes=[pltpu.VMEM((8, 128), jnp.float32)],
    )(x, batch2d)


def _elu(x):
    return jnp.where(x > 0, x, jnp.exp(jnp.minimum(x, 0.0)) - 1.0)


def _gln_norm_elu(x, batch2d, stats, w, b):
    N, C = x.shape

    def body(x_ref, b_ref, st_ref, w_ref, bb_ref, o_ref):
        onehot = (b_ref[...] == lax.broadcasted_iota(
            jnp.int32, (MMB, 128), 1)).astype(jnp.float32)
        ms = jnp.dot(onehot, st_ref[...], preferred_element_type=jnp.float32)
        xn = (x_ref[...] - ms[:, :1]) * ms[:, 1:2] * w_ref[0] + bb_ref[0]
        o_ref[...] = _elu(xn)

    return pl.pallas_call(
        body,
        grid=(N // MMB,),
        in_specs=[pl.BlockSpec((MMB, C), lambda i: (i, 0)),
                  pl.BlockSpec((MMB, 1), lambda i: (i, 0)),
                  pl.BlockSpec((128, 2), lambda i: (0, 0)),
                  pl.BlockSpec((1, C), lambda i: (0, 0)),
                  pl.BlockSpec((1, C), lambda i: (0, 0))],
        out_specs=pl.BlockSpec((MMB, C), lambda i: (i, 0)),
        out_shape=jax.ShapeDtypeStruct((N, C), jnp.float32),
    )(x, batch2d, stats, w, b)


def _score_exp(rel, root_b, batch2d):
    N = root_b.shape[0]

    def body(rel_ref, rt_ref, b_ref, e_ref, d_ref, s_ref):
        i = pl.program_id(0)

        @pl.when(i == 0)
        def _():
            s_ref[...] = jnp.zeros_like(s_ref)

        score = rel_ref[...].sum(axis=0) + rt_ref[...]
        e = jnp.exp(score)
        e_ref[...] = e
        onehot = (b_ref[...] == lax.broadcasted_iota(
            jnp.int32, (MMB, 128), 1)).astype(jnp.float32)
        s_ref[0, :] += (onehot * e).sum(axis=0)
        d_ref[0, :] = s_ref[0, :]

    return pl.pallas_call(
        body,
        grid=(N // MMB,),
        in_specs=[pl.BlockSpec((32, MMB, 1), lambda i: (0, i, 0)),
                  pl.BlockSpec((MMB, 1), lambda i: (i, 0)),
                  pl.BlockSpec((MMB, 1), lambda i: (i, 0))],
        out_specs=[pl.BlockSpec((MMB, 1), lambda i: (i, 0)),
                   pl.BlockSpec((1, 128), lambda i: (0, 0))],
        out_shape=[jax.ShapeDtypeStruct((N, 1), jnp.float32),
                   jax.ShapeDtypeStruct((1, 128), jnp.float32)],
        scratch_shapes=[pltpu.VMEM((8, 128), jnp.float32)],
    )(rel, root_b, batch2d)


def _aa_final(e, dsum, aa_h, aa_x, batch2d):
    N, C = aa_h.shape

    def body(e_ref, d_ref, h_ref, x_ref, b_ref, o_ref, g_ref, acc):
        i = pl.program_id(0)

        @pl.when(i == 0)
        def _():
            acc[...] = jnp.zeros_like(acc)

        onehot = (b_ref[...] == lax.broadcasted_iota(
            jnp.int32, (MMB, 128), 1)).astype(jnp.float32)
        drow = jnp.dot(onehot, d_ref[...], preferred_element_type=jnp.float32)
        score = e_ref[...] / (drow + 1e-16)
        out = x_ref[...] + _elu(h_ref[...] * score)
        o_ref[...] = out
        acc[...] += lax.dot_general(onehot, out, (((0,), (0,)), ((), ())),
                                    preferred_element_type=jnp.float32)
        g_ref[...] = acc[...]

    return pl.pallas_call(
        body,
        grid=(N // MMB,),
        in_specs=[pl.BlockSpec((MMB, 1), lambda i: (i, 0)),
                  pl.BlockSpec((128, 1), lambda i: (0, 0)),
                  pl.BlockSpec((MMB, C), lambda i: (i, 0)),
                  pl.BlockSpec((MMB, C), lambda i: (i, 0)),
                  pl.BlockSpec((MMB, 1), lambda i: (i, 0))],
        out_specs=[pl.BlockSpec((MMB, C), lambda i: (i, 0)),
                   pl.BlockSpec((128, C), lambda i: (0, 0))],
        out_shape=[jax.ShapeDtypeStruct((N, C), jnp.float32),
                   jax.ShapeDtypeStruct((128, C), jnp.float32)],
        scratch_shapes=[pltpu.VMEM((128, C), jnp.float32)],
    )(e, dsum, aa_h, aa_x, batch2d)


def _atom_final(p_lo, p_hi, bias, atom_x, batch2d):
    N = atom_x.shape[0]
    C = 200

    def body(lo_ref, hi_ref, b_ref, x_ref, bt_ref, o_ref, g_ref, acc):
        i = pl.program_id(0)

        @pl.when(i == 0)
        def _():
            acc[...] = jnp.zeros_like(acc)

        lo = lo_ref[0] + lo_ref[1]
        hi = hi_ref[0] + hi_ref[1]
        pooled = jnp.concatenate([lo, hi[:, :72]], axis=1)
        d = hi[:, 72:77]
        dexp = jnp.broadcast_to(d[:, :, None], (MMB, NH, 40)).reshape(MMB, 200)
        pooled = pooled / (dexp + 1e-16) + b_ref[0, :200]
        out = x_ref[...] + _elu(pooled)
        o_ref[...] = out
        onehot = (bt_ref[...] == lax.broadcasted_iota(
            jnp.int32, (MMB, 128), 1)).astype(jnp.float32)
        acc[...] += lax.dot_general(onehot, out, (((0,), (0,)), ((), ())),
                                    preferred_element_type=jnp.float32)
        g_ref[...] = acc[...]

    bs = pl.BlockSpec((2, MMB, 128), lambda i: (0, i, 0))
    return pl.pallas_call(
        body,
        grid=(N // MMB,),
        in_specs=[bs, bs,
                  pl.BlockSpec((1, 256), lambda i: (0, 0)),
                  pl.BlockSpec((MMB, C), lambda i: (i, 0)),
                  pl.BlockSpec((MMB, 1), lambda i: (i, 0))],
        out_specs=[pl.BlockSpec((MMB, C), lambda i: (i, 0)),
                   pl.BlockSpec((128, C), lambda i: (0, 0))],
        out_shape=[jax.ShapeDtypeStruct((N, C), jnp.float32),
                   jax.ShapeDtypeStruct((128, C), jnp.float32)],
        scratch_shapes=[pltpu.VMEM((128, C), jnp.float32)],
    )(p_lo, p_hi, bias, atom_x, batch2d)


def _fold_att(W, att):
    H, C = att.shape
    return (W.reshape(-1, H, C) * att[None]).sum(-1)


def _pad_cols(W, width):
    return jnp.pad(W, ((0, 0), (0, width - W.shape[1])))


def _group_edges(idx, E):
    return jnp.pad(idx.astype(jnp.int32).reshape(E // 16, 16),
                   ((0, CHUNK), (0, 0)))


def kernel(atom_x, atom_edge_index, bond_x, atom_batch, aa_x, aa_edge_index,
           aa_edge_attr, aa_batch, m2p_edge_index, params):
    E_ATOM = atom_edge_index.shape[1]
    E_AA = aa_edge_index.shape[1]
    E_M2P = m2p_edge_index.shape[1]

    pD = params['drug_conv']
    pP = params['prot_conv']
    pI = params['inter_conv']
    pL = params['drug_pool']

    sa_g = _group_edges(atom_edge_index[0], E_ATOM)
    da_g = _group_edges(atom_edge_index[1], E_ATOM)
    sp_g = _group_edges(aa_edge_index[0], E_AA)
    dp_g = _group_edges(aa_edge_index[1], E_AA)
    ms_g = _group_edges(m2p_edge_index[0], E_M2P)
    mp_g = _group_edges(m2p_edge_index[1], E_M2P)

    batch_a = atom_batch.astype(jnp.int32).reshape(-1, 1)
    batch_p = aa_batch.astype(jnp.int32).reshape(-1, 1)

    def ones_cols(h):
        return h.at[:, 100:105].set(1.0)

    W_atom = jnp.concatenate([
        _pad_cols(pD['W_src'], RW),
        _fold_att(pD['W_src'], pD['att_src']),
        _fold_att(pD['W_dst'], pD['att_dst']),
        _fold_att(pI['W_dst'], pI['att_dst']),
    ], axis=1)
    acat = _mm(atom_x, W_atom)
    hs_drug = ones_cols(acat[:, :RW])
    av_drug = acat[:, 128:133].reshape(-1)
    ad_drug = acat[:, 133:138].reshape(-1)
    ad_int_atom = acat[:, 138:143].reshape(-1)

    W_aa = jnp.concatenate([
        _pad_cols(pP['W_src'], RW),
        _fold_att(pP['W_src'], pP['att_src']),
        _fold_att(pP['W_dst'], pP['att_dst']),
        _pad_cols(pI['W_src'], RW),
        _fold_att(pI['W_src'], pI['att_src']),
        _fold_att(pI['W_dst'], pI['att_dst']),
    ], axis=1)
    pcat = _mm(aa_x, W_aa)
    hs_prot = ones_cols(pcat[:, :RW])
    av_prot = pcat[:, 128:133].reshape(-1)
    ad_prot = pcat[:, 133:138].reshape(-1)
    hs_int_a = ones_cols(pcat[:, 138:266])
    av_int_a = pcat[:, 266:271].reshape(-1)
    ad_int_aa = pcat[:, 271:276].reshape(-1)

    me_drug = _pad_cols(_fold_att(pD['W_edge'], pD['att_edge']), 8)
    me_prot = _pad_cols(_fold_att(pP['W_edge'], pP['att_edge']), 8)
    me_pool = _pad_cols(_fold_att(pL['W_edge'], pL['att_edge']), 8)
    ae_drug = jnp.pad(_mm(bond_x, me_drug),
                      ((0, CHUNK * 16), (0, 0))).reshape(-1)
    ae_prot = jnp.pad(_mm(aa_edge_attr, me_prot),
                      ((0, CHUNK * 16), (0, 0))).reshape(-1)
    ae_pool = jnp.pad(_mm(bond_x, me_pool),
                      ((0, CHUNK * 16), (0, 0))).reshape(-1)

    def gat_edge(hs_full, av, ad, ae, srcg, dstg, E, C, PW):
        if ae is not None:
            sp = _make_sc_logits(E, True, False)(av, ae, srcg)
        else:
            sp = _make_sc_logits(E, False, False)(av, srcg)
        s8 = _make_sc_logits(E, True, True)(ad, sp, dstg)
        src_f = srcg.reshape(-1)
        dst_f = dstg.reshape(-1)
        outs = []
        for j in range(hs_full.shape[1] // 128):
            hs_j = hs_full[:, 128 * j:128 * (j + 1)]
            lo = _make_sc_pay(E, C, 128 * j, PW, True)(hs_j, s8, src_f, dst_f)
            hi = _make_sc_pay(E, C, 128 * j, PW, False)(hs_j, s8, src_f, dst_f)
            outs.append(jnp.concatenate([lo, hi], axis=1))
        return outs

    accA = gat_edge(hs_drug, av_drug, ad_drug, ae_drug, sa_g, da_g,
                    E_ATOM, 20, 100)[0]
    accB = gat_edge(hs_int_a, av_int_a, ad_int_atom, None, mp_g, ms_g,
                    E_M2P, 20, 100)[0]

    biasD = _pad_cols(pD['bias'].reshape(1, -1), 128)
    biasI = _pad_cols(pI['bias'].reshape(1, -1), 128)
    xcat_a = _finalize_concat(accA, accB, biasD, biasI)
    stats_a = _gln_stats(xcat_a, batch_a)
    atom_h = _gln_norm_elu(xcat_a, batch_a, stats_a.T,
                           params['drug_norm_w'].reshape(1, -1),
                           params['drug_norm_b'].reshape(1, -1))

    W_ah = jnp.concatenate([
        _pad_cols(pL['W_src'], 256),
        _fold_att(pL['W_src'], pL['att_src']),
        _fold_att(pL['W_dst'], pL['att_dst']),
        _pad_cols(pI['W_src'], 128),
        _fold_att(pI['W_src'], pI['att_src']),
    ], axis=1)
    hcat = _mm(atom_h, W_ah)
    hs_pool = hcat[:, :256].at[:, 200:205].set(1.0)
    av_pool = hcat[:, 256:261].reshape(-1)
    ad_pool = hcat[:, 261:266].reshape(-1)
    hs_int_h = ones_cols(hcat[:, 266:394])
    av_int_h = hcat[:, 394:399].reshape(-1)

    accC = gat_edge(hs_prot, av_prot, ad_prot, ae_prot, sp_g, dp_g,
                    E_AA, 20, 100)[0]
    accD = gat_edge(hs_int_h, av_int_h, ad_int_aa, None, ms_g, mp_g,
                    E_M2P, 20, 100)[0]
    accP = gat_edge(hs_pool, av_pool, ad_pool, ae_pool, sa_g, da_g,
                    E_ATOM, 40, 200)

    biasP = _pad_cols(pP['bias'].reshape(1, -1), 128)
    xcat_p = _finalize_concat(accC, accD, biasP, biasI)
    stats_p = _gln_stats(xcat_p, batch_p)
    aa_h = _gln_norm_elu(xcat_p, batch_p, stats_p.T,
                         params['prot_norm_w'].reshape(1, -1),
                         params['prot_norm_b'].reshape(1, -1))

    W_sag = jnp.concatenate([
        _pad_cols(params['sag_W_rel'], 4), _pad_cols(params['sag_W_root'], 4),
    ], axis=1)
    tr = _mm(aa_h, W_sag)
    t = tr[:, 0]
    root_b = tr[:, 4:5] + params['sag_b_rel'][0]

    rel = _make_sc_seg(E_AA)(t, sp_g, dp_g).reshape(32, 10240)
    rel3 = rel[:, :N_NODE, None]

    e_s, dsum = _score_exp(rel3, root_b, batch_p)
    aa_out, prot_g = _aa_final(e_s, dsum.reshape(128, 1), aa_h, aa_x, batch_p)

    biasL = _pad_cols(pL['bias'].reshape(1, -1), 256)
    atom_out, drug_g = _atom_final(accP[0], accP[1], biasL, atom_x, batch_a)

    return (atom_out, aa_out, drug_g, prot_g)


# speedup vs baseline: 1.5351x; 1.5351x over previous
import dataclasses
import functools
import math

import jax
import jax.numpy as jnp
from jax import lax
from jax.experimental import pallas as pl
from jax.experimental.pallas import tpu as pltpu
from jax.experimental.pallas import tpu_sc as plsc

N_NODE = 10000
RW = 128
NH = 5
CHUNK = 64
ROWS_T = 632


def _cdiv(a, b):
    return (a + b - 1) // b


_T_GATHER = True
_T_SCALE = True
_T_SCAT = True
_T_DEN = True


def _sc_params():
    cp = pltpu.CompilerParams()
    if "needs_layout_passes" in pltpu.CompilerParams.__dataclass_fields__:
        cp = dataclasses.replace(cp, needs_layout_passes=False)
    return cp


def _worker_range(wid, GT, chunk):
    U = _cdiv(GT, chunk)
    ub, ur = divmod(U, 32)
    u0 = wid * ub + jnp.minimum(wid, ur)
    nu = ub + jnp.where(wid < ur, 1, 0)
    g0 = u0 * chunk
    ng = jnp.minimum(nu * chunk, jnp.maximum(GT - g0, 0))
    max_chunks = ub + 1
    return g0, ng, max_chunks


CHUNK_A = 32


def _make_sc_logits(E, has_prev, is_dst):
    GT = E // 16
    mesh = plsc.VectorSubcoreMesh(core_axis_name="c", subcore_axis_name="s")

    scratch = dict(
        tab_v=pltpu.VMEM((N_NODE * 5,), jnp.float32),
        idx_c=pltpu.VMEM((CHUNK_A, 16), jnp.int32),
        s_st=pltpu.VMEM((CHUNK_A * 128,), jnp.float32),
    )
    if has_prev:
        scratch['pv_c'] = pltpu.VMEM((CHUNK_A * 128,), jnp.float32)

    def body(*refs):
        names = ['tab'] + (['prev'] if has_prev else []) + ['idxg', 's_o']
        names += list(scratch.keys())
        r = dict(zip(names, refs))

        cid = lax.axis_index("c")
        sid = lax.axis_index("s")
        wid = sid * 2 + cid
        iota = lax.iota(jnp.int32, 16)
        zv = jnp.zeros((16,), jnp.float32)

        pltpu.sync_copy(r['tab'], r['tab_v'])

        g0, ng, max_chunks = _worker_range(wid, GT, CHUNK_A)

        @pl.loop(0, max_chunks)
        def _(ci):
            @pl.when(ci * CHUNK_A < ng)
            def _():
                gbase = g0 + ci * CHUNK_A
                glen = jnp.minimum(CHUNK_A, ng - ci * CHUNK_A)
                pltpu.sync_copy(r['idxg'].at[pl.ds(gbase, CHUNK_A), :],
                                r['idx_c'])
                if has_prev:
                    pltpu.sync_copy(
                        r['prev'].at[pl.ds(gbase * 128, CHUNK_A * 128)],
                        r['pv_c'])

                @pl.loop(0, CHUNK_A)
                def _(gi):
                    nodev = r['idx_c'][gi]
                    for h in range(NH):
                        a = plsc.load_gather(r['tab_v'], [nodev * 5 + h])
                        if has_prev:
                            a = a + plsc.load_gather(
                                r['pv_c'], [gi * 128 + iota * 8 + h])
                        if is_dst:
                            a = jnp.maximum(a, 0.2 * a)
                            a = jnp.exp(a)
                        plsc.store_scatter(
                            r['s_st'], [gi * 128 + iota * 8 + h], a)
                    for c in range(5, 8):
                        plsc.store_scatter(
                            r['s_st'], [gi * 128 + iota * 8 + c], zv)
                pltpu.sync_copy(
                    r['s_st'],
                    r['s_o'].at[pl.ds(gbase * 128, CHUNK_A * 128)])

    return pl.kernel(
        body,
        out_type=jax.ShapeDtypeStruct(((GT + CHUNK) * 128,), jnp.float32),
        mesh=mesh, compiler_params=_sc_params(),
        scratch_types=list(scratch.values()))


CHUNK_B = 32
MGG = 4


def _make_sc_pay(E, C, col_off, PW, lo):
    GT = E // 16
    SPLIT = 8128
    SH_ROWS = 8192 if lo else 2048
    REAL = SPLIT if lo else N_NODE - SPLIT
    RPT = 512 if lo else 120
    NMG = CHUNK_B // MGG
    mesh = plsc.VectorSubcoreMesh(core_axis_name="c", subcore_axis_name="s")

    scratch = dict(
        src_f=pltpu.VMEM((CHUNK_B * 16,), jnp.int32),
        dst_f=pltpu.VMEM((CHUNK_B * 16,), jnp.int32),
        dst_c2=pltpu.VMEM((NMG, 64), jnp.int32),
        s_c=pltpu.VMEM((CHUNK_B * 128,), jnp.float32),
        rows0=pltpu.VMEM((64, 128), jnp.float32),
        rows1=pltpu.VMEM((64, 128), jnp.float32),
        gsem0=pltpu.SemaphoreType.DMA,
        gsem1=pltpu.SemaphoreType.DMA,
        ssem0=pltpu.SemaphoreType.DMA,
        ssem1=pltpu.SemaphoreType.DMA,
        acc_sh=pltpu.VMEM_SHARED((SH_ROWS, 128), jnp.float32),
    )

    def body(hs, s8, src_h, dst_h, acc_o, src_f, dst_f, dst_c2, s_c,
             rows0, rows1, gsem0, gsem1, ssem0, ssem1, acc_sh):
        cid = lax.axis_index("c")
        sid = lax.axis_index("s")
        wid = sid * 2 + cid
        iota = lax.iota(jnp.int32, 16)
        zv = jnp.zeros((16,), jnp.float32)
        rows = [rows0, rows1]
        gsem = [gsem0, gsem1]
        ssem = [ssem0, ssem1]

        @pl.loop(0, 64)
        def _(i):
            for k in range(8):
                rows0[i, pl.ds(16 * k, 16)] = zv

        z0 = sid * (SH_ROWS // 16)
        nzc = _cdiv(SH_ROWS // 16, 64)

        @pl.loop(0, nzc)
        def _(czi):
            start = jnp.minimum(z0 + czi * 64, z0 + SH_ROWS // 16 - 64)
            pltpu.sync_copy(rows0, acc_sh.at[pl.ds(start, 64), :])

        plsc.subcore_barrier()

        g0, ng, max_chunks = _worker_range(wid, GT, CHUNK_B)

        hvecs = []
        for k in range(8):
            gcol = iota + (16 * k + col_off)
            hv = jnp.where(
                gcol < PW, gcol // C,
                jnp.where(gcol < PW + 5, gcol - PW, 7))
            hvecs.append(hv)

        @pl.loop(0, max_chunks)
        def _(ci):
            @pl.when(ci * CHUNK_B < ng)
            def _():
                gbase = g0 + ci * CHUNK_B
                glen = jnp.minimum(CHUNK_B, ng - ci * CHUNK_B)
                pltpu.sync_copy(src_h.at[pl.ds(gbase * 16, CHUNK_B * 16)],
                                src_f)
                pltpu.sync_copy(dst_h.at[pl.ds(gbase * 16, CHUNK_B * 16)],
                                dst_f)
                pltpu.sync_copy(s8.at[pl.ds(gbase * 128, CHUNK_B * 128)], s_c)

                @pl.loop(0, CHUNK_B)
                def _(i):
                    dstv = dst_f[pl.ds(i * 16, 16)]
                    if lo:
                        dstv = jnp.where(dstv < SPLIT, dstv, SPLIT + sid)
                    else:
                        dstv = jnp.where(dstv >= SPLIT, dstv - SPLIT,
                                         REAL + sid)
                    dstv = jnp.where(i < glen, dstv,
                                     (SPLIT if lo else REAL) + sid)
                    dst_c2[i // MGG, pl.ds((i % MGG) * 16, 16)] = dstv

                def scale(buf, mg):
                    @pl.loop(0, 64)
                    def _(rr):
                        sv = plsc.load_gather(
                            s_c, [(mg * 64 + rr) * 8 + (iota & 7)])
                        for k in range(8):
                            sc = sv.at[hvecs[k]].get(
                                mode='promise_in_bounds')
                            buf[rr, pl.ds(16 * k, 16)] = (
                                buf[rr, pl.ds(16 * k, 16)] * sc)

                def gath(mg, bi):
                    return pltpu.async_copy(
                        hs.at[src_f.at[pl.ds(mg * 64, 64)]], rows[bi],
                        gsem[bi])

                def scat(mg, bi):
                    return pltpu.async_copy(
                        rows[bi], acc_sh.at[dst_c2.at[mg]], ssem[bi],
                        add=True)

                pend_g = {0: gath(0, 0), 1: gath(1, 1)}
                pend_s = {}
                for mg in range(NMG):
                    bi = mg & 1
                    pend_g[mg].wait()
                    scale(rows[bi], mg)
                    if mg >= 1:
                        bp = 1 - bi
                        pend_s[mg - 1].wait()
                        if mg + 1 < NMG:
                            pend_g[mg + 1] = gath(mg + 1, bp)
                    pend_s[mg] = scat(mg, bi)
                pend_s[NMG - 1].wait()

        plsc.subcore_barrier()

        r0 = sid * RPT
        rtop = jnp.minimum(r0 + RPT, REAL) - 64
        noc = _cdiv(RPT, 64)

        @pl.loop(0, noc)
        def _(co):
            start = jnp.minimum(r0 + co * 64, rtop)
            pltpu.sync_copy(acc_sh.at[pl.ds(start, 64), :], rows0)
            pltpu.sync_copy(rows0, acc_o.at[cid, pl.ds(start, 64), :])

    return pl.kernel(
        body,
        out_type=jax.ShapeDtypeStruct((2, REAL, 128), jnp.float32),
        mesh=mesh, compiler_params=_sc_params(),
        scratch_types=list(scratch.values()))


def _make_sc_seg(E):
    GT = E // 16
    U = _cdiv(GT, 8)
    ub, ur = divmod(U, 32)
    max_chunks = _cdiv((ub + 1) * 8, CHUNK)
    NP = 10240

    mesh = plsc.VectorSubcoreMesh(core_axis_name="c", subcore_axis_name="s")

    def body(t_hbm, srcg, dstg, rel_o, t_v, src_c, dst_c, part_v):
        cid = lax.axis_index("c")
        sid = lax.axis_index("s")
        wid = sid * 2 + cid
        zv = jnp.zeros((16,), jnp.float32)

        @pl.loop(0, NP // 16)
        def _(i):
            part_v[pl.ds(i * 16, 16)] = zv

        pltpu.sync_copy(t_hbm, t_v)

        u0 = wid * ub + jnp.minimum(wid, ur)
        nu = ub + jnp.where(wid < ur, 1, 0)
        g0 = u0 * 8
        ng = jnp.minimum(nu * 8, GT - g0)

        @pl.loop(0, max_chunks)
        def _(ci):
            @pl.when(ci * CHUNK < ng)
            def _():
                gbase = g0 + ci * CHUNK
                glen = jnp.minimum(CHUNK, ng - ci * CHUNK)
                pltpu.sync_copy(srcg.at[pl.ds(gbase, CHUNK), :], src_c)
                pltpu.sync_copy(dstg.at[pl.ds(gbase, CHUNK), :], dst_c)

                @pl.loop(0, CHUNK)
                def _(gi):
                    @pl.when(gi < glen)
                    def _():
                        srcv = src_c[gi]
                        dstv = dst_c[gi]
                        t16 = plsc.load_gather(t_v, [srcv])
                        plsc.addupdate_scatter(part_v, [dstv], t16)

        pltpu.sync_copy(part_v, rel_o.at[pl.ds(wid * NP, NP)])

    return pl.kernel(
        body,
        out_type=jax.ShapeDtypeStruct((32 * NP,), jnp.float32),
        mesh=mesh,
        compiler_params=_sc_params(),
        scratch_types=[
            pltpu.VMEM((N_NODE,), jnp.float32),
            pltpu.VMEM((CHUNK, 16), jnp.int32),
            pltpu.VMEM((CHUNK, 16), jnp.int32),
            pltpu.VMEM((NP,), jnp.float32),
        ])


MMB = 400


def _mm(x, w):
    N, D = x.shape
    K = w.shape[1]

    def body(x_ref, w_ref, o_ref):
        o_ref[...] = jnp.dot(x_ref[...], w_ref[...],
                             preferred_element_type=jnp.float32)

    return pl.pallas_call(
        body,
        grid=(N // MMB,),
        in_specs=[pl.BlockSpec((MMB, D), lambda i: (i, 0)),
                  pl.BlockSpec((D, K), lambda i: (0, 0))],
        out_specs=pl.BlockSpec((MMB, K), lambda i: (i, 0)),
        out_shape=jax.ShapeDtypeStruct((N, K), jnp.float32),
    )(x, w)


def _finalize_concat(accA, accB, biasA, biasB):
    N = accA.shape[1]

    def body(aA, aB, bA, bB, o_ref):
        def half(a_ref, b_ref):
            a = a_ref[0] + a_ref[1]
            d = a[:, 100:105]
            dexp = jnp.broadcast_to(d[:, :, None], (MMB, NH, 100 // NH))
            dexp = dexp.reshape(MMB, 100)
            return a[:, :100] / (dexp + 1e-16) + b_ref[0, :100]
        o_ref[...] = jnp.concatenate([half(aA, bA), half(aB, bB)], axis=1)

    bs = pl.BlockSpec((2, MMB, 128), lambda i: (0, i, 0))
    return pl.pallas_call(
        body,
        grid=(N // MMB,),
        in_specs=[bs, bs,
                  pl.BlockSpec((1, 128), lambda i: (0, 0)),
                  pl.BlockSpec((1, 128), lambda i: (0, 0))],
        out_specs=pl.BlockSpec((MMB, 200), lambda i: (i, 0)),
        out_shape=jax.ShapeDtypeStruct((N, 200), jnp.float32),
    )(accA, accB, biasA, biasB)


def _gln_stats(x, batch2d):
    N, C = x.shape
    NB = N // MMB

    def body(x_ref, b_ref, o_ref, s_ref):
        i = pl.program_id(0)

        @pl.when(i == 0)
        def _():
            s_ref[...] = jnp.zeros_like(s_ref)

        xb = x_ref[...]
        onehot = (b_ref[...] == lax.broadcasted_iota(
            jnp.int32, (MMB, 128), 1)).astype(jnp.float32)
        s_ref[0, :] += (onehot * xb.sum(axis=1, keepdims=True)).sum(axis=0)
        s_ref[1, :] += (onehot * (xb * xb).sum(axis=1, keepdims=True)).sum(axis=0)
        s_ref[2, :] += onehot.sum(axis=0)

        cnt = jnp.maximum(s_ref[2, :] * C, 1.0)
        mean = s_ref[0, :] / cnt
        var = s_ref[1, :] / cnt - mean * mean
        o_ref[0, :] = mean
        o_ref[1, :] = lax.rsqrt(var + 1e-5)

    return pl.pallas_call(
        body,
        grid=(NB,),
        in_specs=[pl.BlockSpec((MMB, C), lambda i: (i, 0)),
                  pl.BlockSpec((MMB, 1), lambda i: (i, 0))],
        out_specs=pl.BlockSpec((2, 128), lambda i: (0, 0)),
        out_shape=jax.ShapeDtypeStruct((2, 128), jnp.float32),
        scratch_shapes=[pltpu.VMEM((8, 128), jnp.float32)],
    )(x, batch2d)


def _elu(x):
    return jnp.where(x > 0, x, jnp.exp(jnp.minimum(x, 0.0)) - 1.0)


def _gln_norm_elu(x, batch2d, stats, w, b):
    N, C = x.shape

    def body(x_ref, b_ref, st_ref, w_ref, bb_ref, o_ref):
        onehot = (b_ref[...] == lax.broadcasted_iota(
            jnp.int32, (MMB, 128), 1)).astype(jnp.float32)
        ms = jnp.dot(onehot, st_ref[...], preferred_element_type=jnp.float32)
        xn = (x_ref[...] - ms[:, :1]) * ms[:, 1:2] * w_ref[0] + bb_ref[0]
        o_ref[...] = _elu(xn)

    return pl.pallas_call(
        body,
        grid=(N // MMB,),
        in_specs=[pl.BlockSpec((MMB, C), lambda i: (i, 0)),
                  pl.BlockSpec((MMB, 1), lambda i: (i, 0)),
                  pl.BlockSpec((128, 2), lambda i: (0, 0)),
                  pl.BlockSpec((1, C), lambda i: (0, 0)),
                  pl.BlockSpec((1, C), lambda i: (0, 0))],
        out_specs=pl.BlockSpec((MMB, C), lambda i: (i, 0)),
        out_shape=jax.ShapeDtypeStruct((N, C), jnp.float32),
    )(x, batch2d, stats, w, b)


def _score_exp(rel, root_b, batch2d):
    N = root_b.shape[0]

    def body(rel_ref, rt_ref, b_ref, e_ref, d_ref, s_ref):
        i = pl.program_id(0)

        @pl.when(i == 0)
        def _():
            s_ref[...] = jnp.zeros_like(s_ref)

        score = rel_ref[...].sum(axis=0) + rt_ref[...]
        e = jnp.exp(score)
        e_ref[...] = e
        onehot = (b_ref[...] == lax.broadcasted_iota(
            jnp.int32, (MMB, 128), 1)).astype(jnp.float32)
        s_ref[0, :] += (onehot * e).sum(axis=0)
        d_ref[0, :] = s_ref[0, :]

    return pl.pallas_call(
        body,
        grid=(N // MMB,),
        in_specs=[pl.BlockSpec((32, MMB, 1), lambda i: (0, i, 0)),
                  pl.BlockSpec((MMB, 1), lambda i: (i, 0)),
                  pl.BlockSpec((MMB, 1), lambda i: (i, 0))],
        out_specs=[pl.BlockSpec((MMB, 1), lambda i: (i, 0)),
                   pl.BlockSpec((1, 128), lambda i: (0, 0))],
        out_shape=[jax.ShapeDtypeStruct((N, 1), jnp.float32),
                   jax.ShapeDtypeStruct((1, 128), jnp.float32)],
        scratch_shapes=[pltpu.VMEM((8, 128), jnp.float32)],
    )(rel, root_b, batch2d)


def _aa_final(e, dsum, aa_h, aa_x, batch2d):
    N, C = aa_h.shape

    def body(e_ref, d_ref, h_ref, x_ref, b_ref, o_ref, g_ref, acc):
        i = pl.program_id(0)

        @pl.when(i == 0)
        def _():
            acc[...] = jnp.zeros_like(acc)

        onehot = (b_ref[...] == lax.broadcasted_iota(
            jnp.int32, (MMB, 128), 1)).astype(jnp.float32)
        drow = jnp.dot(onehot, d_ref[...], preferred_element_type=jnp.float32)
        score = e_ref[...] / (drow + 1e-16)
        out = x_ref[...] + _elu(h_ref[...] * score)
        o_ref[...] = out
        acc[...] += lax.dot_general(onehot, out, (((0,), (0,)), ((), ())),
                                    preferred_element_type=jnp.float32)
        g_ref[...] = acc[...]

    return pl.pallas_call(
        body,
        grid=(N // MMB,),
        in_specs=[pl.BlockSpec((MMB, 1), lambda i: (i, 0)),
                  pl.BlockSpec((128, 1), lambda i: (0, 0)),
                  pl.BlockSpec((MMB, C), lambda i: (i, 0)),
                  pl.BlockSpec((MMB, C), lambda i: (i, 0)),
                  pl.BlockSpec((MMB, 1), lambda i: (i, 0))],
        out_specs=[pl.BlockSpec((MMB, C), lambda i: (i, 0)),
                   pl.BlockSpec((128, C), lambda i: (0, 0))],
        out_shape=[jax.ShapeDtypeStruct((N, C), jnp.float32),
                   jax.ShapeDtypeStruct((128, C), jnp.float32)],
        scratch_shapes=[pltpu.VMEM((128, C), jnp.float32)],
    )(e, dsum, aa_h, aa_x, batch2d)


def _atom_final(p_lo, p_hi, bias, atom_x, batch2d):
    N = atom_x.shape[0]
    C = 200

    def body(lo_ref, hi_ref, b_ref, x_ref, bt_ref, o_ref, g_ref, acc):
        i = pl.program_id(0)

        @pl.when(i == 0)
        def _():
            acc[...] = jnp.zeros_like(acc)

        lo = lo_ref[0] + lo_ref[1]
        hi = hi_ref[0] + hi_ref[1]
        pooled = jnp.concatenate([lo, hi[:, :72]], axis=1)
        d = hi[:, 72:77]
        dexp = jnp.broadcast_to(d[:, :, None], (MMB, NH, 40)).reshape(MMB, 200)
        pooled = pooled / (dexp + 1e-16) + b_ref[0, :200]
        out = x_ref[...] + _elu(pooled)
        o_ref[...] = out
        onehot = (bt_ref[...] == lax.broadcasted_iota(
            jnp.int32, (MMB, 128), 1)).astype(jnp.float32)
        acc[...] += lax.dot_general(onehot, out, (((0,), (0,)), ((), ())),
                                    preferred_element_type=jnp.float32)
        g_ref[...] = acc[...]

    bs = pl.BlockSpec((2, MMB, 128), lambda i: (0, i, 0))
    return pl.pallas_call(
        body,
        grid=(N // MMB,),
        in_specs=[bs, bs,
                  pl.BlockSpec((1, 256), lambda i: (0, 0)),
                  pl.BlockSpec((MMB, C), lambda i: (i, 0)),
                  pl.BlockSpec((MMB, 1), lambda i: (i, 0))],
        out_specs=[pl.BlockSpec((MMB, C), lambda i: (i, 0)),
                   pl.BlockSpec((128, C), lambda i: (0, 0))],
        out_shape=[jax.ShapeDtypeStruct((N, C), jnp.float32),
                   jax.ShapeDtypeStruct((128, C), jnp.float32)],
        scratch_shapes=[pltpu.VMEM((128, C), jnp.float32)],
    )(p_lo, p_hi, bias, atom_x, batch2d)


def _fold_att(W, att):
    H, C = att.shape
    return (W.reshape(-1, H, C) * att[None]).sum(-1)


def _pad_cols(W, width):
    return jnp.pad(W, ((0, 0), (0, width - W.shape[1])))


def _group_edges(idx, E):
    return jnp.pad(idx.astype(jnp.int32).reshape(E // 16, 16),
                   ((0, CHUNK), (0, 0)))


def kernel(atom_x, atom_edge_index, bond_x, atom_batch, aa_x, aa_edge_index,
           aa_edge_attr, aa_batch, m2p_edge_index, params):
    E_ATOM = atom_edge_index.shape[1]
    E_AA = aa_edge_index.shape[1]
    E_M2P = m2p_edge_index.shape[1]

    pD = params['drug_conv']
    pP = params['prot_conv']
    pI = params['inter_conv']
    pL = params['drug_pool']

    sa_g = _group_edges(atom_edge_index[0], E_ATOM)
    da_g = _group_edges(atom_edge_index[1], E_ATOM)
    sp_g = _group_edges(aa_edge_index[0], E_AA)
    dp_g = _group_edges(aa_edge_index[1], E_AA)
    ms_g = _group_edges(m2p_edge_index[0], E_M2P)
    mp_g = _group_edges(m2p_edge_index[1], E_M2P)

    batch_a = atom_batch.astype(jnp.int32).reshape(-1, 1)
    batch_p = aa_batch.astype(jnp.int32).reshape(-1, 1)

    def ones_cols(h):
        return h.at[:, 100:105].set(1.0)

    W_atom = jnp.concatenate([
        _pad_cols(pD['W_src'], RW),
        _fold_att(pD['W_src'], pD['att_src']),
        _fold_att(pD['W_dst'], pD['att_dst']),
        _fold_att(pI['W_dst'], pI['att_dst']),
    ], axis=1)
    acat = _mm(atom_x, W_atom)
    hs_drug = ones_cols(acat[:, :RW])
    av_drug = acat[:, 128:133].reshape(-1)
    ad_drug = acat[:, 133:138].reshape(-1)
    ad_int_atom = acat[:, 138:143].reshape(-1)

    W_aa = jnp.concatenate([
        _pad_cols(pP['W_src'], RW),
        _fold_att(pP['W_src'], pP['att_src']),
        _fold_att(pP['W_dst'], pP['att_dst']),
        _pad_cols(pI['W_src'], RW),
        _fold_att(pI['W_src'], pI['att_src']),
        _fold_att(pI['W_dst'], pI['att_dst']),
    ], axis=1)
    pcat = _mm(aa_x, W_aa)
    hs_prot = ones_cols(pcat[:, :RW])
    av_prot = pcat[:, 128:133].reshape(-1)
    ad_prot = pcat[:, 133:138].reshape(-1)
    hs_int_a = ones_cols(pcat[:, 138:266])
    av_int_a = pcat[:, 266:271].reshape(-1)
    ad_int_aa = pcat[:, 271:276].reshape(-1)

    me_drug = _pad_cols(_fold_att(pD['W_edge'], pD['att_edge']), 8)
    me_prot = _pad_cols(_fold_att(pP['W_edge'], pP['att_edge']), 8)
    me_pool = _pad_cols(_fold_att(pL['W_edge'], pL['att_edge']), 8)
    ae_drug = jnp.pad(_mm(bond_x, me_drug),
                      ((0, CHUNK * 16), (0, 0))).reshape(-1)
    ae_prot = jnp.pad(_mm(aa_edge_attr, me_prot),
                      ((0, CHUNK * 16), (0, 0))).reshape(-1)
    ae_pool = jnp.pad(_mm(bond_x, me_pool),
                      ((0, CHUNK * 16), (0, 0))).reshape(-1)

    def gat_edge(hs_full, av, ad, ae, srcg, dstg, E, C, PW):
        if ae is not None:
            sp = _make_sc_logits(E, True, False)(av, ae, srcg)
        else:
            sp = _make_sc_logits(E, False, False)(av, srcg)
        s8 = _make_sc_logits(E, True, True)(ad, sp, dstg)
        src_f = srcg.reshape(-1)
        dst_f = dstg.reshape(-1)
        outs = []
        for j in range(hs_full.shape[1] // 128):
            hs_j = hs_full[:, 128 * j:128 * (j + 1)]
            lo = _make_sc_pay(E, C, 128 * j, PW, True)(hs_j, s8, src_f, dst_f)
            hi = _make_sc_pay(E, C, 128 * j, PW, False)(hs_j, s8, src_f, dst_f)
            outs.append(jnp.concatenate([lo, hi], axis=1))
        return outs

    accA = gat_edge(hs_drug, av_drug, ad_drug, ae_drug, sa_g, da_g,
                    E_ATOM, 20, 100)[0]
    accB = gat_edge(hs_int_a, av_int_a, ad_int_atom, None, mp_g, ms_g,
                    E_M2P, 20, 100)[0]

    biasD = _pad_cols(pD['bias'].reshape(1, -1), 128)
    biasI = _pad_cols(pI['bias'].reshape(1, -1), 128)
    xcat_a = _finalize_concat(accA, accB, biasD, biasI)
    stats_a = _gln_stats(xcat_a, batch_a)
    atom_h = _gln_norm_elu(xcat_a, batch_a, stats_a.T,
                           params['drug_norm_w'].reshape(1, -1),
                           params['drug_norm_b'].reshape(1, -1))

    W_ah = jnp.concatenate([
        _pad_cols(pL['W_src'], 256),
        _fold_att(pL['W_src'], pL['att_src']),
        _fold_att(pL['W_dst'], pL['att_dst']),
        _pad_cols(pI['W_src'], 128),
        _fold_att(pI['W_src'], pI['att_src']),
    ], axis=1)
    hcat = _mm(atom_h, W_ah)
    hs_pool = hcat[:, :256].at[:, 200:205].set(1.0)
    av_pool = hcat[:, 256:261].reshape(-1)
    ad_pool = hcat[:, 261:266].reshape(-1)
    hs_int_h = ones_cols(hcat[:, 266:394])
    av_int_h = hcat[:, 394:399].reshape(-1)

    accC = gat_edge(hs_prot, av_prot, ad_prot, ae_prot, sp_g, dp_g,
                    E_AA, 20, 100)[0]
    accD = gat_edge(hs_int_h, av_int_h, ad_int_aa, None, ms_g, mp_g,
                    E_M2P, 20, 100)[0]
    accP = gat_edge(hs_pool, av_pool, ad_pool, ae_pool, sa_g, da_g,
                    E_ATOM, 40, 200)

    biasP = _pad_cols(pP['bias'].reshape(1, -1), 128)
    xcat_p = _finalize_concat(accC, accD, biasP, biasI)
    stats_p = _gln_stats(xcat_p, batch_p)
    aa_h = _gln_norm_elu(xcat_p, batch_p, stats_p.T,
                         params['prot_norm_w'].reshape(1, -1),
                         params['prot_norm_b'].reshape(1, -1))

    W_sag = jnp.concatenate([
        _pad_cols(params['sag_W_rel'], 4), _pad_cols(params['sag_W_root'], 4),
    ], axis=1)
    tr = _mm(aa_h, W_sag)
    t = tr[:, 0]
    root_b = tr[:, 4:5] + params['sag_b_rel'][0]

    rel = _make_sc_seg(E_AA)(t, sp_g, dp_g).reshape(32, 10240)
    rel3 = rel[:, :N_NODE, None]

    e_s, dsum = _score_exp(rel3, root_b, batch_p)
    aa_out, prot_g = _aa_final(e_s, dsum.reshape(128, 1), aa_h, aa_x, batch_p)

    biasL = _pad_cols(pL['bias'].reshape(1, -1), 256)
    atom_out, drug_g = _atom_final(accP[0], accP[1], biasL, atom_x, batch_a)

    return (atom_out, aa_out, drug_g, prot_g)


# speedup vs baseline: 1.5606x; 1.0166x over previous
import dataclasses
import functools
import math

import jax
import jax.numpy as jnp
from jax import lax
from jax.experimental import pallas as pl
from jax.experimental.pallas import tpu as pltpu
from jax.experimental.pallas import tpu_sc as plsc

N_NODE = 10000
RW = 128
NH = 5
CHUNK = 64
ROWS_T = 632


def _cdiv(a, b):
    return (a + b - 1) // b


_T_GATHER = True
_T_SCALE = True
_T_SCAT = True
_T_DEN = True


def _sc_params():
    cp = pltpu.CompilerParams()
    if "needs_layout_passes" in pltpu.CompilerParams.__dataclass_fields__:
        cp = dataclasses.replace(cp, needs_layout_passes=False)
    return cp


def _worker_range(wid, GT, chunk, nw=32):
    U = _cdiv(GT, chunk)
    ub, ur = divmod(U, nw)
    u0 = wid * ub + jnp.minimum(wid, ur)
    nu = ub + jnp.where(wid < ur, 1, 0)
    g0 = u0 * chunk
    ng = jnp.minimum(nu * chunk, jnp.maximum(GT - g0, 0))
    max_chunks = ub + 1
    return g0, ng, max_chunks


CHUNK_A = 32


def _make_sc_logits(E, has_prev, is_dst):
    GT = E // 16
    mesh = plsc.VectorSubcoreMesh(core_axis_name="c", subcore_axis_name="s")

    scratch = dict(
        tab_v=pltpu.VMEM((N_NODE * 5,), jnp.float32),
        idx_c=pltpu.VMEM((CHUNK_A, 16), jnp.int32),
        s_st=pltpu.VMEM((CHUNK_A * 128,), jnp.float32),
    )
    if has_prev:
        scratch['pv_c'] = pltpu.VMEM((CHUNK_A * 128,), jnp.float32)

    def body(*refs):
        names = ['tab'] + (['prev'] if has_prev else []) + ['idxg', 's_o']
        names += list(scratch.keys())
        r = dict(zip(names, refs))

        cid = lax.axis_index("c")
        sid = lax.axis_index("s")
        wid = sid * 2 + cid
        iota = lax.iota(jnp.int32, 16)
        zv = jnp.zeros((16,), jnp.float32)

        pltpu.sync_copy(r['tab'], r['tab_v'])

        g0, ng, max_chunks = _worker_range(wid, GT, CHUNK_A)

        @pl.loop(0, max_chunks)
        def _(ci):
            @pl.when(ci * CHUNK_A < ng)
            def _():
                gbase = g0 + ci * CHUNK_A
                glen = jnp.minimum(CHUNK_A, ng - ci * CHUNK_A)
                pltpu.sync_copy(r['idxg'].at[pl.ds(gbase, CHUNK_A), :],
                                r['idx_c'])
                if has_prev:
                    pltpu.sync_copy(
                        r['prev'].at[pl.ds(gbase * 128, CHUNK_A * 128)],
                        r['pv_c'])

                @pl.loop(0, CHUNK_A)
                def _(gi):
                    nodev = r['idx_c'][gi]
                    for h in range(NH):
                        a = plsc.load_gather(r['tab_v'], [nodev * 5 + h])
                        if has_prev:
                            a = a + plsc.load_gather(
                                r['pv_c'], [gi * 128 + iota * 8 + h])
                        if is_dst:
                            a = jnp.maximum(a, 0.2 * a)
                            a = jnp.exp(a)
                        plsc.store_scatter(
                            r['s_st'], [gi * 128 + iota * 8 + h], a)
                    for c in range(5, 8):
                        plsc.store_scatter(
                            r['s_st'], [gi * 128 + iota * 8 + c], zv)
                pltpu.sync_copy(
                    r['s_st'],
                    r['s_o'].at[pl.ds(gbase * 128, CHUNK_A * 128)])

    return pl.kernel(
        body,
        out_type=jax.ShapeDtypeStruct(((GT + CHUNK) * 128,), jnp.float32),
        mesh=mesh, compiler_params=_sc_params(),
        scratch_types=list(scratch.values()))


CHUNK_B = 32
MGG = 4
SPLIT = 8128


def _make_sc_pay(E, C, col_off, PW):
    GT = E // 16
    NMG = CHUNK_B // MGG
    mesh = plsc.VectorSubcoreMesh(core_axis_name="c", subcore_axis_name="s")

    scratch = dict(
        src_f=pltpu.VMEM((CHUNK_B * 16,), jnp.int32),
        dst_f=pltpu.VMEM((CHUNK_B * 16,), jnp.int32),
        dst_c2=pltpu.VMEM((NMG, 64), jnp.int32),
        s_c=pltpu.VMEM((CHUNK_B * 128,), jnp.float32),
        rows0=pltpu.VMEM((64, 128), jnp.float32),
        rows1=pltpu.VMEM((64, 128), jnp.float32),
        gsem0=pltpu.SemaphoreType.DMA,
        gsem1=pltpu.SemaphoreType.DMA,
        ssem0=pltpu.SemaphoreType.DMA,
        ssem1=pltpu.SemaphoreType.DMA,
        acc_sh=pltpu.VMEM_SHARED((8192, 128), jnp.float32),
    )

    def body(hs, s8, src_h, dst_h, acc_o, src_f, dst_f, dst_c2, s_c,
             rows0, rows1, gsem0, gsem1, ssem0, ssem1, acc_sh):
        cid = lax.axis_index("c")
        sid = lax.axis_index("s")
        iota = lax.iota(jnp.int32, 16)
        zv = jnp.zeros((16,), jnp.float32)
        rows = [rows0, rows1]
        gsem = [gsem0, gsem1]
        ssem = [ssem0, ssem1]

        is_lo = cid == 0
        real = jnp.where(is_lo, SPLIT, N_NODE - SPLIT)
        rpt = jnp.where(is_lo, 512, 120)
        zpt = jnp.where(is_lo, 512, 128)

        @pl.loop(0, 64)
        def _(i):
            for k in range(8):
                rows0[i, pl.ds(16 * k, 16)] = zv

        z0 = sid * zpt

        @pl.loop(0, 8)
        def _(czi):
            start = jnp.minimum(z0 + czi * 64, z0 + zpt - 64)
            pltpu.sync_copy(rows0, acc_sh.at[pl.ds(start, 64), :])

        plsc.subcore_barrier()

        g0, ng, max_chunks = _worker_range(sid, GT, CHUNK_B, nw=16)

        hvecs = []
        for k in range(8):
            gcol = iota + (16 * k + col_off)
            hv = jnp.where(
                gcol < PW, gcol // C,
                jnp.where(gcol < PW + 5, gcol - PW, 7))
            hvecs.append(hv)

        @pl.loop(0, max_chunks)
        def _(ci):
            @pl.when(ci * CHUNK_B < ng)
            def _():
                gbase = g0 + ci * CHUNK_B
                glen = jnp.minimum(CHUNK_B, ng - ci * CHUNK_B)
                pltpu.sync_copy(src_h.at[pl.ds(gbase * 16, CHUNK_B * 16)],
                                src_f)
                pltpu.sync_copy(dst_h.at[pl.ds(gbase * 16, CHUNK_B * 16)],
                                dst_f)
                pltpu.sync_copy(s8.at[pl.ds(gbase * 128, CHUNK_B * 128)], s_c)

                garb = real + sid

                @pl.loop(0, CHUNK_B)
                def _(i):
                    dstv = dst_f[pl.ds(i * 16, 16)]
                    dstv = jnp.where(
                        is_lo,
                        jnp.where(dstv < SPLIT, dstv, garb),
                        jnp.where(dstv >= SPLIT, dstv - SPLIT, garb))
                    dstv = jnp.where(i < glen, dstv, garb)
                    dst_c2[i // MGG, pl.ds((i % MGG) * 16, 16)] = dstv

                def scale(buf, mg):
                    @pl.loop(0, 64)
                    def _(rr):
                        sv = plsc.load_gather(
                            s_c, [(mg * 64 + rr) * 8 + (iota & 7)])
                        for k in range(8):
                            sc = sv.at[hvecs[k]].get(
                                mode='promise_in_bounds')
                            buf[rr, pl.ds(16 * k, 16)] = (
                                buf[rr, pl.ds(16 * k, 16)] * sc)

                def gath(mg, bi):
                    return pltpu.async_copy(
                        hs.at[src_f.at[pl.ds(mg * 64, 64)]], rows[bi],
                        gsem[bi])

                def scat(mg, bi):
                    return pltpu.async_copy(
                        rows[bi], acc_sh.at[dst_c2.at[mg]], ssem[bi],
                        add=True)

                pend_g = {0: gath(0, 0), 1: gath(1, 1)}
                pend_s = {}
                for mg in range(NMG):
                    bi = mg & 1
                    pend_g[mg].wait()
                    scale(rows[bi], mg)
                    if mg >= 1:
                        bp = 1 - bi
                        pend_s[mg - 1].wait()
                        if mg + 1 < NMG:
                            pend_g[mg + 1] = gath(mg + 1, bp)
                    pend_s[mg] = scat(mg, bi)
                pend_s[NMG - 1].wait()

        plsc.subcore_barrier()

        r0 = sid * rpt
        rtop = jnp.minimum(r0 + rpt, real) - 64

        @pl.loop(0, 8)
        def _(co):
            start = jnp.minimum(r0 + co * 64, rtop)
            pltpu.sync_copy(acc_sh.at[pl.ds(start, 64), :], rows0)
            pltpu.sync_copy(rows0, acc_o.at[cid, pl.ds(start, 64), :])

    return pl.kernel(
        body,
        out_type=jax.ShapeDtypeStruct((2, SPLIT, 128), jnp.float32),
        mesh=mesh, compiler_params=_sc_params(),
        scratch_types=list(scratch.values()))


def _make_sc_seg(E):
    GT = E // 16
    U = _cdiv(GT, 8)
    ub, ur = divmod(U, 32)
    max_chunks = _cdiv((ub + 1) * 8, CHUNK)
    NP = 10240

    mesh = plsc.VectorSubcoreMesh(core_axis_name="c", subcore_axis_name="s")

    def body(t_hbm, srcg, dstg, rel_o, t_v, src_c, dst_c, part_v):
        cid = lax.axis_index("c")
        sid = lax.axis_index("s")
        wid = sid * 2 + cid
        zv = jnp.zeros((16,), jnp.float32)

        @pl.loop(0, NP // 16)
        def _(i):
            part_v[pl.ds(i * 16, 16)] = zv

        pltpu.sync_copy(t_hbm, t_v)

        u0 = wid * ub + jnp.minimum(wid, ur)
        nu = ub + jnp.where(wid < ur, 1, 0)
        g0 = u0 * 8
        ng = jnp.minimum(nu * 8, GT - g0)

        @pl.loop(0, max_chunks)
        def _(ci):
            @pl.when(ci * CHUNK < ng)
            def _():
                gbase = g0 + ci * CHUNK
                glen = jnp.minimum(CHUNK, ng - ci * CHUNK)
                pltpu.sync_copy(srcg.at[pl.ds(gbase, CHUNK), :], src_c)
                pltpu.sync_copy(dstg.at[pl.ds(gbase, CHUNK), :], dst_c)

                @pl.loop(0, CHUNK)
                def _(gi):
                    @pl.when(gi < glen)
                    def _():
                        srcv = src_c[gi]
                        dstv = dst_c[gi]
                        t16 = plsc.load_gather(t_v, [srcv])
                        plsc.addupdate_scatter(part_v, [dstv], t16)

        pltpu.sync_copy(part_v, rel_o.at[pl.ds(wid * NP, NP)])

    return pl.kernel(
        body,
        out_type=jax.ShapeDtypeStruct((32 * NP,), jnp.float32),
        mesh=mesh,
        compiler_params=_sc_params(),
        scratch_types=[
            pltpu.VMEM((N_NODE,), jnp.float32),
            pltpu.VMEM((CHUNK, 16), jnp.int32),
            pltpu.VMEM((CHUNK, 16), jnp.int32),
            pltpu.VMEM((NP,), jnp.float32),
        ])


MMB = 400


def _mm(x, w):
    N, D = x.shape
    K = w.shape[1]

    def body(x_ref, w_ref, o_ref):
        o_ref[...] = jnp.dot(x_ref[...], w_ref[...],
                             preferred_element_type=jnp.float32)

    return pl.pallas_call(
        body,
        grid=(N // MMB,),
        in_specs=[pl.BlockSpec((MMB, D), lambda i: (i, 0)),
                  pl.BlockSpec((D, K), lambda i: (0, 0))],
        out_specs=pl.BlockSpec((MMB, K), lambda i: (i, 0)),
        out_shape=jax.ShapeDtypeStruct((N, K), jnp.float32),
    )(x, w)


def _finalize_concat(accA, accB, biasA, biasB):
    N = accA.shape[0]

    def body(aA, aB, bA, bB, o_ref):
        def half(a_ref, b_ref):
            a = a_ref[...]
            d = a[:, 100:105]
            dexp = jnp.broadcast_to(d[:, :, None], (MMB, NH, 100 // NH))
            dexp = dexp.reshape(MMB, 100)
            return a[:, :100] / (dexp + 1e-16) + b_ref[0, :100]
        o_ref[...] = jnp.concatenate([half(aA, bA), half(aB, bB)], axis=1)

    bs = pl.BlockSpec((MMB, 128), lambda i: (i, 0))
    return pl.pallas_call(
        body,
        grid=(N // MMB,),
        in_specs=[bs, bs,
                  pl.BlockSpec((1, 128), lambda i: (0, 0)),
                  pl.BlockSpec((1, 128), lambda i: (0, 0))],
        out_specs=pl.BlockSpec((MMB, 200), lambda i: (i, 0)),
        out_shape=jax.ShapeDtypeStruct((N, 200), jnp.float32),
    )(accA, accB, biasA, biasB)


def _gln_stats(x, batch2d):
    N, C = x.shape
    NB = N // MMB

    def body(x_ref, b_ref, o_ref, s_ref):
        i = pl.program_id(0)

        @pl.when(i == 0)
        def _():
            s_ref[...] = jnp.zeros_like(s_ref)

        xb = x_ref[...]
        onehot = (b_ref[...] == lax.broadcasted_iota(
            jnp.int32, (MMB, 128), 1)).astype(jnp.float32)
        s_ref[0, :] += (onehot * xb.sum(axis=1, keepdims=True)).sum(axis=0)
        s_ref[1, :] += (onehot * (xb * xb).sum(axis=1, keepdims=True)).sum(axis=0)
        s_ref[2, :] += onehot.sum(axis=0)

        cnt = jnp.maximum(s_ref[2, :] * C, 1.0)
        mean = s_ref[0, :] / cnt
        var = s_ref[1, :] / cnt - mean * mean
        o_ref[0, :] = mean
        o_ref[1, :] = lax.rsqrt(var + 1e-5)

    return pl.pallas_call(
        body,
        grid=(NB,),
        in_specs=[pl.BlockSpec((MMB, C), lambda i: (i, 0)),
                  pl.BlockSpec((MMB, 1), lambda i: (i, 0))],
        out_specs=pl.BlockSpec((2, 128), lambda i: (0, 0)),
        out_shape=jax.ShapeDtypeStruct((2, 128), jnp.float32),
        scratch_shapes=[pltpu.VMEM((8, 128), jnp.float32)],
    )(x, batch2d)


def _elu(x):
    return jnp.where(x > 0, x, jnp.exp(jnp.minimum(x, 0.0)) - 1.0)


def _gln_norm_elu(x, batch2d, stats, w, b):
    N, C = x.shape

    def body(x_ref, b_ref, st_ref, w_ref, bb_ref, o_ref):
        onehot = (b_ref[...] == lax.broadcasted_iota(
            jnp.int32, (MMB, 128), 1)).astype(jnp.float32)
        ms = jnp.dot(onehot, st_ref[...], preferred_element_type=jnp.float32)
        xn = (x_ref[...] - ms[:, :1]) * ms[:, 1:2] * w_ref[0] + bb_ref[0]
        o_ref[...] = _elu(xn)

    return pl.pallas_call(
        body,
        grid=(N // MMB,),
        in_specs=[pl.BlockSpec((MMB, C), lambda i: (i, 0)),
                  pl.BlockSpec((MMB, 1), lambda i: (i, 0)),
                  pl.BlockSpec((128, 2), lambda i: (0, 0)),
                  pl.BlockSpec((1, C), lambda i: (0, 0)),
                  pl.BlockSpec((1, C), lambda i: (0, 0))],
        out_specs=pl.BlockSpec((MMB, C), lambda i: (i, 0)),
        out_shape=jax.ShapeDtypeStruct((N, C), jnp.float32),
    )(x, batch2d, stats, w, b)


def _score_exp(rel, root_b, batch2d):
    N = root_b.shape[0]

    def body(rel_ref, rt_ref, b_ref, e_ref, d_ref, s_ref):
        i = pl.program_id(0)

        @pl.when(i == 0)
        def _():
            s_ref[...] = jnp.zeros_like(s_ref)

        score = rel_ref[...].sum(axis=0) + rt_ref[...]
        e = jnp.exp(score)
        e_ref[...] = e
        onehot = (b_ref[...] == lax.broadcasted_iota(
            jnp.int32, (MMB, 128), 1)).astype(jnp.float32)
        s_ref[0, :] += (onehot * e).sum(axis=0)
        d_ref[0, :] = s_ref[0, :]

    return pl.pallas_call(
        body,
        grid=(N // MMB,),
        in_specs=[pl.BlockSpec((32, MMB, 1), lambda i: (0, i, 0)),
                  pl.BlockSpec((MMB, 1), lambda i: (i, 0)),
                  pl.BlockSpec((MMB, 1), lambda i: (i, 0))],
        out_specs=[pl.BlockSpec((MMB, 1), lambda i: (i, 0)),
                   pl.BlockSpec((1, 128), lambda i: (0, 0))],
        out_shape=[jax.ShapeDtypeStruct((N, 1), jnp.float32),
                   jax.ShapeDtypeStruct((1, 128), jnp.float32)],
        scratch_shapes=[pltpu.VMEM((8, 128), jnp.float32)],
    )(rel, root_b, batch2d)


def _aa_final(e, dsum, aa_h, aa_x, batch2d):
    N, C = aa_h.shape

    def body(e_ref, d_ref, h_ref, x_ref, b_ref, o_ref, g_ref, acc):
        i = pl.program_id(0)

        @pl.when(i == 0)
        def _():
            acc[...] = jnp.zeros_like(acc)

        onehot = (b_ref[...] == lax.broadcasted_iota(
            jnp.int32, (MMB, 128), 1)).astype(jnp.float32)
        drow = jnp.dot(onehot, d_ref[...], preferred_element_type=jnp.float32)
        score = e_ref[...] / (drow + 1e-16)
        out = x_ref[...] + _elu(h_ref[...] * score)
        o_ref[...] = out
        acc[...] += lax.dot_general(onehot, out, (((0,), (0,)), ((), ())),
                                    preferred_element_type=jnp.float32)
        g_ref[...] = acc[...]

    return pl.pallas_call(
        body,
        grid=(N // MMB,),
        in_specs=[pl.BlockSpec((MMB, 1), lambda i: (i, 0)),
                  pl.BlockSpec((128, 1), lambda i: (0, 0)),
                  pl.BlockSpec((MMB, C), lambda i: (i, 0)),
                  pl.BlockSpec((MMB, C), lambda i: (i, 0)),
                  pl.BlockSpec((MMB, 1), lambda i: (i, 0))],
        out_specs=[pl.BlockSpec((MMB, C), lambda i: (i, 0)),
                   pl.BlockSpec((128, C), lambda i: (0, 0))],
        out_shape=[jax.ShapeDtypeStruct((N, C), jnp.float32),
                   jax.ShapeDtypeStruct((128, C), jnp.float32)],
        scratch_shapes=[pltpu.VMEM((128, C), jnp.float32)],
    )(e, dsum, aa_h, aa_x, batch2d)


def _atom_final(p_lo, p_hi, bias, atom_x, batch2d):
    N = atom_x.shape[0]
    C = 200

    def body(lo_ref, hi_ref, b_ref, x_ref, bt_ref, o_ref, g_ref, acc):
        i = pl.program_id(0)

        @pl.when(i == 0)
        def _():
            acc[...] = jnp.zeros_like(acc)

        lo = lo_ref[...]
        hi = hi_ref[...]
        pooled = jnp.concatenate([lo, hi[:, :72]], axis=1)
        d = hi[:, 72:77]
        dexp = jnp.broadcast_to(d[:, :, None], (MMB, NH, 40)).reshape(MMB, 200)
        pooled = pooled / (dexp + 1e-16) + b_ref[0, :200]
        out = x_ref[...] + _elu(pooled)
        o_ref[...] = out
        onehot = (bt_ref[...] == lax.broadcasted_iota(
            jnp.int32, (MMB, 128), 1)).astype(jnp.float32)
        acc[...] += lax.dot_general(onehot, out, (((0,), (0,)), ((), ())),
                                    preferred_element_type=jnp.float32)
        g_ref[...] = acc[...]

    bs = pl.BlockSpec((MMB, 128), lambda i: (i, 0))
    return pl.pallas_call(
        body,
        grid=(N // MMB,),
        in_specs=[bs, bs,
                  pl.BlockSpec((1, 256), lambda i: (0, 0)),
                  pl.BlockSpec((MMB, C), lambda i: (i, 0)),
                  pl.BlockSpec((MMB, 1), lambda i: (i, 0))],
        out_specs=[pl.BlockSpec((MMB, C), lambda i: (i, 0)),
                   pl.BlockSpec((128, C), lambda i: (0, 0))],
        out_shape=[jax.ShapeDtypeStruct((N, C), jnp.float32),
                   jax.ShapeDtypeStruct((128, C), jnp.float32)],
        scratch_shapes=[pltpu.VMEM((128, C), jnp.float32)],
    )(p_lo, p_hi, bias, atom_x, batch2d)


def _fold_att(W, att):
    H, C = att.shape
    return (W.reshape(-1, H, C) * att[None]).sum(-1)


def _pad_cols(W, width):
    return jnp.pad(W, ((0, 0), (0, width - W.shape[1])))


def _group_edges(idx, E):
    return jnp.pad(idx.astype(jnp.int32).reshape(E // 16, 16),
                   ((0, CHUNK), (0, 0)))


def kernel(atom_x, atom_edge_index, bond_x, atom_batch, aa_x, aa_edge_index,
           aa_edge_attr, aa_batch, m2p_edge_index, params):
    E_ATOM = atom_edge_index.shape[1]
    E_AA = aa_edge_index.shape[1]
    E_M2P = m2p_edge_index.shape[1]

    pD = params['drug_conv']
    pP = params['prot_conv']
    pI = params['inter_conv']
    pL = params['drug_pool']

    sa_g = _group_edges(atom_edge_index[0], E_ATOM)
    da_g = _group_edges(atom_edge_index[1], E_ATOM)
    sp_g = _group_edges(aa_edge_index[0], E_AA)
    dp_g = _group_edges(aa_edge_index[1], E_AA)
    ms_g = _group_edges(m2p_edge_index[0], E_M2P)
    mp_g = _group_edges(m2p_edge_index[1], E_M2P)

    batch_a = atom_batch.astype(jnp.int32).reshape(-1, 1)
    batch_p = aa_batch.astype(jnp.int32).reshape(-1, 1)

    def ones_cols(h):
        return h.at[:, 100:105].set(1.0)

    W_atom = jnp.concatenate([
        _pad_cols(pD['W_src'], RW),
        _fold_att(pD['W_src'], pD['att_src']),
        _fold_att(pD['W_dst'], pD['att_dst']),
        _fold_att(pI['W_dst'], pI['att_dst']),
    ], axis=1)
    acat = _mm(atom_x, W_atom)
    hs_drug = ones_cols(acat[:, :RW])
    av_drug = acat[:, 128:133].reshape(-1)
    ad_drug = acat[:, 133:138].reshape(-1)
    ad_int_atom = acat[:, 138:143].reshape(-1)

    W_aa = jnp.concatenate([
        _pad_cols(pP['W_src'], RW),
        _fold_att(pP['W_src'], pP['att_src']),
        _fold_att(pP['W_dst'], pP['att_dst']),
        _pad_cols(pI['W_src'], RW),
        _fold_att(pI['W_src'], pI['att_src']),
        _fold_att(pI['W_dst'], pI['att_dst']),
    ], axis=1)
    pcat = _mm(aa_x, W_aa)
    hs_prot = ones_cols(pcat[:, :RW])
    av_prot = pcat[:, 128:133].reshape(-1)
    ad_prot = pcat[:, 133:138].reshape(-1)
    hs_int_a = ones_cols(pcat[:, 138:266])
    av_int_a = pcat[:, 266:271].reshape(-1)
    ad_int_aa = pcat[:, 271:276].reshape(-1)

    me_drug = _pad_cols(_fold_att(pD['W_edge'], pD['att_edge']), 8)
    me_prot = _pad_cols(_fold_att(pP['W_edge'], pP['att_edge']), 8)
    me_pool = _pad_cols(_fold_att(pL['W_edge'], pL['att_edge']), 8)
    ae_drug = jnp.pad(_mm(bond_x, me_drug),
                      ((0, CHUNK * 16), (0, 0))).reshape(-1)
    ae_prot = jnp.pad(_mm(aa_edge_attr, me_prot),
                      ((0, CHUNK * 16), (0, 0))).reshape(-1)
    ae_pool = jnp.pad(_mm(bond_x, me_pool),
                      ((0, CHUNK * 16), (0, 0))).reshape(-1)

    def gat_edge(hs_full, av, ad, ae, srcg, dstg, E, C, PW):
        if ae is not None:
            sp = _make_sc_logits(E, True, False)(av, ae, srcg)
        else:
            sp = _make_sc_logits(E, False, False)(av, srcg)
        s8 = _make_sc_logits(E, True, True)(ad, sp, dstg)
        src_f = srcg.reshape(-1)
        dst_f = dstg.reshape(-1)
        outs = []
        for j in range(hs_full.shape[1] // 128):
            hs_j = hs_full[:, 128 * j:128 * (j + 1)]
            acc = _make_sc_pay(E, C, 128 * j, PW)(hs_j, s8, src_f, dst_f)
            outs.append(jnp.concatenate(
                [acc[0], acc[1, :N_NODE - SPLIT]], axis=0))
        return outs

    accA = gat_edge(hs_drug, av_drug, ad_drug, ae_drug, sa_g, da_g,
                    E_ATOM, 20, 100)[0]
    accB = gat_edge(hs_int_a, av_int_a, ad_int_atom, None, mp_g, ms_g,
                    E_M2P, 20, 100)[0]

    biasD = _pad_cols(pD['bias'].reshape(1, -1), 128)
    biasI = _pad_cols(pI['bias'].reshape(1, -1), 128)
    xcat_a = _finalize_concat(accA, accB, biasD, biasI)
    stats_a = _gln_stats(xcat_a, batch_a)
    atom_h = _gln_norm_elu(xcat_a, batch_a, stats_a.T,
                           params['drug_norm_w'].reshape(1, -1),
                           params['drug_norm_b'].reshape(1, -1))

    W_ah = jnp.concatenate([
        _pad_cols(pL['W_src'], 256),
        _fold_att(pL['W_src'], pL['att_src']),
        _fold_att(pL['W_dst'], pL['att_dst']),
        _pad_cols(pI['W_src'], 128),
        _fold_att(pI['W_src'], pI['att_src']),
    ], axis=1)
    hcat = _mm(atom_h, W_ah)
    hs_pool = hcat[:, :256].at[:, 200:205].set(1.0)
    av_pool = hcat[:, 256:261].reshape(-1)
    ad_pool = hcat[:, 261:266].reshape(-1)
    hs_int_h = ones_cols(hcat[:, 266:394])
    av_int_h = hcat[:, 394:399].reshape(-1)

    accC = gat_edge(hs_prot, av_prot, ad_prot, ae_prot, sp_g, dp_g,
                    E_AA, 20, 100)[0]
    accD = gat_edge(hs_int_h, av_int_h, ad_int_aa, None, ms_g, mp_g,
                    E_M2P, 20, 100)[0]
    accP = gat_edge(hs_pool, av_pool, ad_pool, ae_pool, sa_g, da_g,
                    E_ATOM, 40, 200)

    biasP = _pad_cols(pP['bias'].reshape(1, -1), 128)
    xcat_p = _finalize_concat(accC, accD, biasP, biasI)
    stats_p = _gln_stats(xcat_p, batch_p)
    aa_h = _gln_norm_elu(xcat_p, batch_p, stats_p.T,
                         params['prot_norm_w'].reshape(1, -1),
                         params['prot_norm_b'].reshape(1, -1))

    W_sag = jnp.concatenate([
        _pad_cols(params['sag_W_rel'], 4), _pad_cols(params['sag_W_root'], 4),
    ], axis=1)
    tr = _mm(aa_h, W_sag)
    t = tr[:, 0]
    root_b = tr[:, 4:5] + params['sag_b_rel'][0]

    rel = _make_sc_seg(E_AA)(t, sp_g, dp_g).reshape(32, 10240)
    rel3 = rel[:, :N_NODE, None]

    e_s, dsum = _score_exp(rel3, root_b, batch_p)
    aa_out, prot_g = _aa_final(e_s, dsum.reshape(128, 1), aa_h, aa_x, batch_p)

    biasL = _pad_cols(pL['bias'].reshape(1, -1), 256)
    atom_out, drug_g = _atom_final(accP[0], accP[1], biasL, atom_x, batch_a)

    return (atom_out, aa_out, drug_g, prot_g)


# speedup vs baseline: 1.5857x; 1.0161x over previous
import dataclasses
import functools
import math

import jax
import jax.numpy as jnp
from jax import lax
from jax.experimental import pallas as pl
from jax.experimental.pallas import tpu as pltpu
from jax.experimental.pallas import tpu_sc as plsc

N_NODE = 10000
RW = 128
NH = 5
CHUNK = 64
ROWS_T = 632


def _cdiv(a, b):
    return (a + b - 1) // b


_T_GATHER = True
_T_SCALE = True
_T_SCAT = True
_T_DEN = True


def _sc_params():
    cp = pltpu.CompilerParams()
    if "needs_layout_passes" in pltpu.CompilerParams.__dataclass_fields__:
        cp = dataclasses.replace(cp, needs_layout_passes=False)
    return cp


def _worker_range(wid, GT, chunk, nw=32):
    U = _cdiv(GT, chunk)
    ub, ur = divmod(U, nw)
    u0 = wid * ub + jnp.minimum(wid, ur)
    nu = ub + jnp.where(wid < ur, 1, 0)
    g0 = u0 * chunk
    ng = jnp.minimum(nu * chunk, jnp.maximum(GT - g0, 0))
    max_chunks = ub + 1
    return g0, ng, max_chunks


CHUNK_A = 32


def _make_sc_logits(E, has_prev, is_dst):
    GT = E // 16
    mesh = plsc.VectorSubcoreMesh(core_axis_name="c", subcore_axis_name="s")

    scratch = dict(
        tab_v=pltpu.VMEM((N_NODE * 5,), jnp.float32),
        idx_c=pltpu.VMEM((CHUNK_A, 16), jnp.int32),
        s_st=pltpu.VMEM((CHUNK_A * 128,), jnp.float32),
        isem=pltpu.SemaphoreType.DMA,
        psem=pltpu.SemaphoreType.DMA,
        osem=pltpu.SemaphoreType.DMA,
    )
    if has_prev:
        scratch['pv_c'] = pltpu.VMEM((CHUNK_A * 128,), jnp.float32)

    def body(*refs):
        names = ['tab'] + (['prev'] if has_prev else []) + ['idxg', 's_o']
        names += list(scratch.keys())
        r = dict(zip(names, refs))

        cid = lax.axis_index("c")
        sid = lax.axis_index("s")
        wid = sid * 2 + cid
        iota = lax.iota(jnp.int32, 16)
        zv = jnp.zeros((16,), jnp.float32)

        pltpu.sync_copy(r['tab'], r['tab_v'])

        g0, ng, max_chunks = _worker_range(wid, GT, CHUNK_A)

        @pl.loop(0, max_chunks)
        def _(ci):
            @pl.when(ci * CHUNK_A < ng)
            def _():
                gbase = g0 + ci * CHUNK_A
                ic = pltpu.async_copy(
                    r['idxg'].at[pl.ds(gbase, CHUNK_A), :], r['idx_c'],
                    r['isem'])
                if has_prev:
                    pc = pltpu.async_copy(
                        r['prev'].at[pl.ds(gbase * 128, CHUNK_A * 128)],
                        r['pv_c'], r['psem'])
                ic.wait()
                if has_prev:
                    pc.wait()
                @pl.when(ci >= 1)
                def _():
                    pltpu.make_async_copy(
                        r['s_st'],
                        r['s_o'].at[pl.ds(gbase * 128, CHUNK_A * 128)],
                        r['osem']).wait()

                @pl.loop(0, CHUNK_A)
                def _(gi):
                    nodev = r['idx_c'][gi]
                    for h in range(NH):
                        a = plsc.load_gather(r['tab_v'], [nodev * 5 + h])
                        if has_prev:
                            a = a + plsc.load_gather(
                                r['pv_c'], [gi * 128 + iota * 8 + h])
                        if is_dst:
                            a = jnp.maximum(a, 0.2 * a)
                            a = jnp.exp(a)
                        plsc.store_scatter(
                            r['s_st'], [gi * 128 + iota * 8 + h], a)
                    for c in range(5, 8):
                        plsc.store_scatter(
                            r['s_st'], [gi * 128 + iota * 8 + c], zv)
                pltpu.async_copy(
                    r['s_st'],
                    r['s_o'].at[pl.ds(gbase * 128, CHUNK_A * 128)],
                    r['osem'])

        @pl.when(ng > 0)
        def _():
            pltpu.make_async_copy(
                r['s_st'], r['s_o'].at[pl.ds(g0 * 128, CHUNK_A * 128)],
                r['osem']).wait()

    return pl.kernel(
        body,
        out_type=jax.ShapeDtypeStruct(((GT + CHUNK) * 128,), jnp.float32),
        mesh=mesh, compiler_params=_sc_params(),
        scratch_types=list(scratch.values()))


CHUNK_B = 32
MGG = 4
SPLIT = 8128


def _make_sc_pay(E, C, col_off, PW):
    GT = E // 16
    NMG = CHUNK_B // MGG
    mesh = plsc.VectorSubcoreMesh(core_axis_name="c", subcore_axis_name="s")

    scratch = dict(
        src_f=pltpu.VMEM((CHUNK_B * 16,), jnp.int32),
        dst_f=pltpu.VMEM((CHUNK_B * 16,), jnp.int32),
        dst_c2=pltpu.VMEM((NMG, 64), jnp.int32),
        s_c=pltpu.VMEM((CHUNK_B * 128,), jnp.float32),
        rows0=pltpu.VMEM((64, 128), jnp.float32),
        rows1=pltpu.VMEM((64, 128), jnp.float32),
        gsem0=pltpu.SemaphoreType.DMA,
        gsem1=pltpu.SemaphoreType.DMA,
        ssem0=pltpu.SemaphoreType.DMA,
        ssem1=pltpu.SemaphoreType.DMA,
        acc_sh=pltpu.VMEM_SHARED((8192, 128), jnp.float32),
    )

    def body(hs, s8, src_h, dst_h, acc_o, src_f, dst_f, dst_c2, s_c,
             rows0, rows1, gsem0, gsem1, ssem0, ssem1, acc_sh):
        cid = lax.axis_index("c")
        sid = lax.axis_index("s")
        iota = lax.iota(jnp.int32, 16)
        zv = jnp.zeros((16,), jnp.float32)
        rows = [rows0, rows1]
        gsem = [gsem0, gsem1]
        ssem = [ssem0, ssem1]

        is_lo = cid == 0
        real = jnp.where(is_lo, SPLIT, N_NODE - SPLIT)
        rpt = jnp.where(is_lo, 512, 120)
        zpt = jnp.where(is_lo, 512, 128)

        @pl.loop(0, 64)
        def _(i):
            for k in range(8):
                rows0[i, pl.ds(16 * k, 16)] = zv

        z0 = sid * zpt

        @pl.loop(0, 8)
        def _(czi):
            start = jnp.minimum(z0 + czi * 64, z0 + zpt - 64)
            pltpu.sync_copy(rows0, acc_sh.at[pl.ds(start, 64), :])

        plsc.subcore_barrier()

        g0, ng, max_chunks = _worker_range(sid, GT, CHUNK_B, nw=16)

        hvecs = []
        for k in range(8):
            gcol = iota + (16 * k + col_off)
            hv = jnp.where(
                gcol < PW, gcol // C,
                jnp.where(gcol < PW + 5, gcol - PW, 7))
            hvecs.append(hv)

        @pl.loop(0, max_chunks)
        def _(ci):
            @pl.when(ci * CHUNK_B < ng)
            def _():
                gbase = g0 + ci * CHUNK_B
                glen = jnp.minimum(CHUNK_B, ng - ci * CHUNK_B)
                pltpu.sync_copy(src_h.at[pl.ds(gbase * 16, CHUNK_B * 16)],
                                src_f)
                pltpu.sync_copy(dst_h.at[pl.ds(gbase * 16, CHUNK_B * 16)],
                                dst_f)
                pltpu.sync_copy(s8.at[pl.ds(gbase * 128, CHUNK_B * 128)], s_c)

                garb = real + sid

                @pl.loop(0, CHUNK_B)
                def _(i):
                    dstv = dst_f[pl.ds(i * 16, 16)]
                    dstv = jnp.where(
                        is_lo,
                        jnp.where(dstv < SPLIT, dstv, garb),
                        jnp.where(dstv >= SPLIT, dstv - SPLIT, garb))
                    dstv = jnp.where(i < glen, dstv, garb)
                    dst_c2[i // MGG, pl.ds((i % MGG) * 16, 16)] = dstv

                def scale(buf, mg):
                    @pl.loop(0, 64)
                    def _(rr):
                        sv = plsc.load_gather(
                            s_c, [(mg * 64 + rr) * 8 + (iota & 7)])
                        for k in range(8):
                            sc = sv.at[hvecs[k]].get(
                                mode='promise_in_bounds')
                            buf[rr, pl.ds(16 * k, 16)] = (
                                buf[rr, pl.ds(16 * k, 16)] * sc)

                def gath(mg, bi):
                    return pltpu.async_copy(
                        hs.at[src_f.at[pl.ds(mg * 64, 64)]], rows[bi],
                        gsem[bi])

                def scat(mg, bi):
                    return pltpu.async_copy(
                        rows[bi], acc_sh.at[dst_c2.at[mg]], ssem[bi],
                        add=True)

                pend_g = {0: gath(0, 0), 1: gath(1, 1)}
                pend_s = {}
                for mg in range(NMG):
                    bi = mg & 1
                    pend_g[mg].wait()
                    scale(rows[bi], mg)
                    if mg >= 1:
                        bp = 1 - bi
                        pend_s[mg - 1].wait()
                        if mg + 1 < NMG:
                            pend_g[mg + 1] = gath(mg + 1, bp)
                    pend_s[mg] = scat(mg, bi)
                pend_s[NMG - 1].wait()

        plsc.subcore_barrier()

        r0 = sid * rpt
        rtop = jnp.minimum(r0 + rpt, real) - 64

        @pl.loop(0, 8)
        def _(co):
            start = jnp.minimum(r0 + co * 64, rtop)
            pltpu.sync_copy(acc_sh.at[pl.ds(start, 64), :], rows0)
            pltpu.sync_copy(rows0, acc_o.at[cid, pl.ds(start, 64), :])

    return pl.kernel(
        body,
        out_type=jax.ShapeDtypeStruct((2, SPLIT, 128), jnp.float32),
        mesh=mesh, compiler_params=_sc_params(),
        scratch_types=list(scratch.values()))


def _make_sc_seg(E):
    GT = E // 16
    U = _cdiv(GT, 8)
    ub, ur = divmod(U, 32)
    max_chunks = _cdiv((ub + 1) * 8, CHUNK)
    NP = 10240

    mesh = plsc.VectorSubcoreMesh(core_axis_name="c", subcore_axis_name="s")

    def body(t_hbm, srcg, dstg, rel_o, t_v, src_c, dst_c, part_v):
        cid = lax.axis_index("c")
        sid = lax.axis_index("s")
        wid = sid * 2 + cid
        zv = jnp.zeros((16,), jnp.float32)

        @pl.loop(0, NP // 16)
        def _(i):
            part_v[pl.ds(i * 16, 16)] = zv

        pltpu.sync_copy(t_hbm, t_v)

        u0 = wid * ub + jnp.minimum(wid, ur)
        nu = ub + jnp.where(wid < ur, 1, 0)
        g0 = u0 * 8
        ng = jnp.minimum(nu * 8, GT - g0)

        @pl.loop(0, max_chunks)
        def _(ci):
            @pl.when(ci * CHUNK < ng)
            def _():
                gbase = g0 + ci * CHUNK
                glen = jnp.minimum(CHUNK, ng - ci * CHUNK)
                pltpu.sync_copy(srcg.at[pl.ds(gbase, CHUNK), :], src_c)
                pltpu.sync_copy(dstg.at[pl.ds(gbase, CHUNK), :], dst_c)

                @pl.loop(0, CHUNK)
                def _(gi):
                    @pl.when(gi < glen)
                    def _():
                        srcv = src_c[gi]
                        dstv = dst_c[gi]
                        t16 = plsc.load_gather(t_v, [srcv])
                        plsc.addupdate_scatter(part_v, [dstv], t16)

        pltpu.sync_copy(part_v, rel_o.at[pl.ds(wid * NP, NP)])

    return pl.kernel(
        body,
        out_type=jax.ShapeDtypeStruct((32 * NP,), jnp.float32),
        mesh=mesh,
        compiler_params=_sc_params(),
        scratch_types=[
            pltpu.VMEM((N_NODE,), jnp.float32),
            pltpu.VMEM((CHUNK, 16), jnp.int32),
            pltpu.VMEM((CHUNK, 16), jnp.int32),
            pltpu.VMEM((NP,), jnp.float32),
        ])


MMB = 400


def _mm(x, w):
    N, D = x.shape
    K = w.shape[1]

    def body(x_ref, w_ref, o_ref):
        o_ref[...] = jnp.dot(x_ref[...], w_ref[...],
                             preferred_element_type=jnp.float32)

    return pl.pallas_call(
        body,
        grid=(N // MMB,),
        in_specs=[pl.BlockSpec((MMB, D), lambda i: (i, 0)),
                  pl.BlockSpec((D, K), lambda i: (0, 0))],
        out_specs=pl.BlockSpec((MMB, K), lambda i: (i, 0)),
        out_shape=jax.ShapeDtypeStruct((N, K), jnp.float32),
    )(x, w)


def _finalize_concat(accA, accB, biasA, biasB):
    N = accA.shape[0]

    def body(aA, aB, bA, bB, o_ref):
        def half(a_ref, b_ref):
            a = a_ref[...]
            d = a[:, 100:105]
            dexp = jnp.broadcast_to(d[:, :, None], (MMB, NH, 100 // NH))
            dexp = dexp.reshape(MMB, 100)
            return a[:, :100] / (dexp + 1e-16) + b_ref[0, :100]
        o_ref[...] = jnp.concatenate([half(aA, bA), half(aB, bB)], axis=1)

    bs = pl.BlockSpec((MMB, 128), lambda i: (i, 0))
    return pl.pallas_call(
        body,
        grid=(N // MMB,),
        in_specs=[bs, bs,
                  pl.BlockSpec((1, 128), lambda i: (0, 0)),
                  pl.BlockSpec((1, 128), lambda i: (0, 0))],
        out_specs=pl.BlockSpec((MMB, 200), lambda i: (i, 0)),
        out_shape=jax.ShapeDtypeStruct((N, 200), jnp.float32),
    )(accA, accB, biasA, biasB)


def _gln_stats(x, batch2d):
    N, C = x.shape
    NB = N // MMB

    def body(x_ref, b_ref, o_ref, s_ref):
        i = pl.program_id(0)

        @pl.when(i == 0)
        def _():
            s_ref[...] = jnp.zeros_like(s_ref)

        xb = x_ref[...]
        onehot = (b_ref[...] == lax.broadcasted_iota(
            jnp.int32, (MMB, 128), 1)).astype(jnp.float32)
        s_ref[0, :] += (onehot * xb.sum(axis=1, keepdims=True)).sum(axis=0)
        s_ref[1, :] += (onehot * (xb * xb).sum(axis=1, keepdims=True)).sum(axis=0)
        s_ref[2, :] += onehot.sum(axis=0)

        cnt = jnp.maximum(s_ref[2, :] * C, 1.0)
        mean = s_ref[0, :] / cnt
        var = s_ref[1, :] / cnt - mean * mean
        o_ref[0, :] = mean
        o_ref[1, :] = lax.rsqrt(var + 1e-5)

    return pl.pallas_call(
        body,
        grid=(NB,),
        in_specs=[pl.BlockSpec((MMB, C), lambda i: (i, 0)),
                  pl.BlockSpec((MMB, 1), lambda i: (i, 0))],
        out_specs=pl.BlockSpec((2, 128), lambda i: (0, 0)),
        out_shape=jax.ShapeDtypeStruct((2, 128), jnp.float32),
        scratch_shapes=[pltpu.VMEM((8, 128), jnp.float32)],
    )(x, batch2d)


def _elu(x):
    return jnp.where(x > 0, x, jnp.exp(jnp.minimum(x, 0.0)) - 1.0)


def _gln_norm_elu(x, batch2d, stats, w, b):
    N, C = x.shape

    def body(x_ref, b_ref, st_ref, w_ref, bb_ref, o_ref):
        onehot = (b_ref[...] == lax.broadcasted_iota(
            jnp.int32, (MMB, 128), 1)).astype(jnp.float32)
        ms = jnp.dot(onehot, st_ref[...], preferred_element_type=jnp.float32)
        xn = (x_ref[...] - ms[:, :1]) * ms[:, 1:2] * w_ref[0] + bb_ref[0]
        o_ref[...] = _elu(xn)

    return pl.pallas_call(
        body,
        grid=(N // MMB,),
        in_specs=[pl.BlockSpec((MMB, C), lambda i: (i, 0)),
                  pl.BlockSpec((MMB, 1), lambda i: (i, 0)),
                  pl.BlockSpec((128, 2), lambda i: (0, 0)),
                  pl.BlockSpec((1, C), lambda i: (0, 0)),
                  pl.BlockSpec((1, C), lambda i: (0, 0))],
        out_specs=pl.BlockSpec((MMB, C), lambda i: (i, 0)),
        out_shape=jax.ShapeDtypeStruct((N, C), jnp.float32),
    )(x, batch2d, stats, w, b)


def _score_exp(rel, root_b, batch2d):
    N = root_b.shape[0]

    def body(rel_ref, rt_ref, b_ref, e_ref, d_ref, s_ref):
        i = pl.program_id(0)

        @pl.when(i == 0)
        def _():
            s_ref[...] = jnp.zeros_like(s_ref)

        score = rel_ref[...].sum(axis=0) + rt_ref[...]
        e = jnp.exp(score)
        e_ref[...] = e
        onehot = (b_ref[...] == lax.broadcasted_iota(
            jnp.int32, (MMB, 128), 1)).astype(jnp.float32)
        s_ref[0, :] += (onehot * e).sum(axis=0)
        d_ref[0, :] = s_ref[0, :]

    return pl.pallas_call(
        body,
        grid=(N // MMB,),
        in_specs=[pl.BlockSpec((32, MMB, 1), lambda i: (0, i, 0)),
                  pl.BlockSpec((MMB, 1), lambda i: (i, 0)),
                  pl.BlockSpec((MMB, 1), lambda i: (i, 0))],
        out_specs=[pl.BlockSpec((MMB, 1), lambda i: (i, 0)),
                   pl.BlockSpec((1, 128), lambda i: (0, 0))],
        out_shape=[jax.ShapeDtypeStruct((N, 1), jnp.float32),
                   jax.ShapeDtypeStruct((1, 128), jnp.float32)],
        scratch_shapes=[pltpu.VMEM((8, 128), jnp.float32)],
    )(rel, root_b, batch2d)


def _aa_final(e, dsum, aa_h, aa_x, batch2d):
    N, C = aa_h.shape

    def body(e_ref, d_ref, h_ref, x_ref, b_ref, o_ref, g_ref, acc):
        i = pl.program_id(0)

        @pl.when(i == 0)
        def _():
            acc[...] = jnp.zeros_like(acc)

        onehot = (b_ref[...] == lax.broadcasted_iota(
            jnp.int32, (MMB, 128), 1)).astype(jnp.float32)
        drow = jnp.dot(onehot, d_ref[...], preferred_element_type=jnp.float32)
        score = e_ref[...] / (drow + 1e-16)
        out = x_ref[...] + _elu(h_ref[...] * score)
        o_ref[...] = out
        acc[...] += lax.dot_general(onehot, out, (((0,), (0,)), ((), ())),
                                    preferred_element_type=jnp.float32)
        g_ref[...] = acc[...]

    return pl.pallas_call(
        body,
        grid=(N // MMB,),
        in_specs=[pl.BlockSpec((MMB, 1), lambda i: (i, 0)),
                  pl.BlockSpec((128, 1), lambda i: (0, 0)),
                  pl.BlockSpec((MMB, C), lambda i: (i, 0)),
                  pl.BlockSpec((MMB, C), lambda i: (i, 0)),
                  pl.BlockSpec((MMB, 1), lambda i: (i, 0))],
        out_specs=[pl.BlockSpec((MMB, C), lambda i: (i, 0)),
                   pl.BlockSpec((128, C), lambda i: (0, 0))],
        out_shape=[jax.ShapeDtypeStruct((N, C), jnp.float32),
                   jax.ShapeDtypeStruct((128, C), jnp.float32)],
        scratch_shapes=[pltpu.VMEM((128, C), jnp.float32)],
    )(e, dsum, aa_h, aa_x, batch2d)


def _atom_final(p_lo, p_hi, bias, atom_x, batch2d):
    N = atom_x.shape[0]
    C = 200

    def body(lo_ref, hi_ref, b_ref, x_ref, bt_ref, o_ref, g_ref, acc):
        i = pl.program_id(0)

        @pl.when(i == 0)
        def _():
            acc[...] = jnp.zeros_like(acc)

        lo = lo_ref[...]
        hi = hi_ref[...]
        pooled = jnp.concatenate([lo, hi[:, :72]], axis=1)
        d = hi[:, 72:77]
        dexp = jnp.broadcast_to(d[:, :, None], (MMB, NH, 40)).reshape(MMB, 200)
        pooled = pooled / (dexp + 1e-16) + b_ref[0, :200]
        out = x_ref[...] + _elu(pooled)
        o_ref[...] = out
        onehot = (bt_ref[...] == lax.broadcasted_iota(
            jnp.int32, (MMB, 128), 1)).astype(jnp.float32)
        acc[...] += lax.dot_general(onehot, out, (((0,), (0,)), ((), ())),
                                    preferred_element_type=jnp.float32)
        g_ref[...] = acc[...]

    bs = pl.BlockSpec((MMB, 128), lambda i: (i, 0))
    return pl.pallas_call(
        body,
        grid=(N // MMB,),
        in_specs=[bs, bs,
                  pl.BlockSpec((1, 256), lambda i: (0, 0)),
                  pl.BlockSpec((MMB, C), lambda i: (i, 0)),
                  pl.BlockSpec((MMB, 1), lambda i: (i, 0))],
        out_specs=[pl.BlockSpec((MMB, C), lambda i: (i, 0)),
                   pl.BlockSpec((128, C), lambda i: (0, 0))],
        out_shape=[jax.ShapeDtypeStruct((N, C), jnp.float32),
                   jax.ShapeDtypeStruct((128, C), jnp.float32)],
        scratch_shapes=[pltpu.VMEM((128, C), jnp.float32)],
    )(p_lo, p_hi, bias, atom_x, batch2d)


def _fold_att(W, att):
    H, C = att.shape
    return (W.reshape(-1, H, C) * att[None]).sum(-1)


def _pad_cols(W, width):
    return jnp.pad(W, ((0, 0), (0, width - W.shape[1])))


def _group_edges(idx, E):
    return jnp.pad(idx.astype(jnp.int32).reshape(E // 16, 16),
                   ((0, CHUNK), (0, 0)))


def kernel(atom_x, atom_edge_index, bond_x, atom_batch, aa_x, aa_edge_index,
           aa_edge_attr, aa_batch, m2p_edge_index, params):
    E_ATOM = atom_edge_index.shape[1]
    E_AA = aa_edge_index.shape[1]
    E_M2P = m2p_edge_index.shape[1]

    pD = params['drug_conv']
    pP = params['prot_conv']
    pI = params['inter_conv']
    pL = params['drug_pool']

    sa_g = _group_edges(atom_edge_index[0], E_ATOM)
    da_g = _group_edges(atom_edge_index[1], E_ATOM)
    sp_g = _group_edges(aa_edge_index[0], E_AA)
    dp_g = _group_edges(aa_edge_index[1], E_AA)
    ms_g = _group_edges(m2p_edge_index[0], E_M2P)
    mp_g = _group_edges(m2p_edge_index[1], E_M2P)

    batch_a = atom_batch.astype(jnp.int32).reshape(-1, 1)
    batch_p = aa_batch.astype(jnp.int32).reshape(-1, 1)

    def ones_cols(h):
        return h.at[:, 100:105].set(1.0)

    W_atom = jnp.concatenate([
        _pad_cols(pD['W_src'], RW),
        _fold_att(pD['W_src'], pD['att_src']),
        _fold_att(pD['W_dst'], pD['att_dst']),
        _fold_att(pI['W_dst'], pI['att_dst']),
    ], axis=1)
    acat = _mm(atom_x, W_atom)
    hs_drug = ones_cols(acat[:, :RW])
    av_drug = acat[:, 128:133].reshape(-1)
    ad_drug = acat[:, 133:138].reshape(-1)
    ad_int_atom = acat[:, 138:143].reshape(-1)

    W_aa = jnp.concatenate([
        _pad_cols(pP['W_src'], RW),
        _fold_att(pP['W_src'], pP['att_src']),
        _fold_att(pP['W_dst'], pP['att_dst']),
        _pad_cols(pI['W_src'], RW),
        _fold_att(pI['W_src'], pI['att_src']),
        _fold_att(pI['W_dst'], pI['att_dst']),
    ], axis=1)
    pcat = _mm(aa_x, W_aa)
    hs_prot = ones_cols(pcat[:, :RW])
    av_prot = pcat[:, 128:133].reshape(-1)
    ad_prot = pcat[:, 133:138].reshape(-1)
    hs_int_a = ones_cols(pcat[:, 138:266])
    av_int_a = pcat[:, 266:271].reshape(-1)
    ad_int_aa = pcat[:, 271:276].reshape(-1)

    me_drug = _pad_cols(_fold_att(pD['W_edge'], pD['att_edge']), 8)
    me_prot = _pad_cols(_fold_att(pP['W_edge'], pP['att_edge']), 8)
    me_pool = _pad_cols(_fold_att(pL['W_edge'], pL['att_edge']), 8)
    ae_drug = jnp.pad(_mm(bond_x, me_drug),
                      ((0, CHUNK * 16), (0, 0))).reshape(-1)
    ae_prot = jnp.pad(_mm(aa_edge_attr, me_prot),
                      ((0, CHUNK * 16), (0, 0))).reshape(-1)
    ae_pool = jnp.pad(_mm(bond_x, me_pool),
                      ((0, CHUNK * 16), (0, 0))).reshape(-1)

    def gat_edge(hs_full, av, ad, ae, srcg, dstg, E, C, PW):
        if ae is not None:
            sp = _make_sc_logits(E, True, False)(av, ae, srcg)
        else:
            sp = _make_sc_logits(E, False, False)(av, srcg)
        s8 = _make_sc_logits(E, True, True)(ad, sp, dstg)
        src_f = srcg.reshape(-1)
        dst_f = dstg.reshape(-1)
        outs = []
        for j in range(hs_full.shape[1] // 128):
            hs_j = hs_full[:, 128 * j:128 * (j + 1)]
            acc = _make_sc_pay(E, C, 128 * j, PW)(hs_j, s8, src_f, dst_f)
            outs.append(jnp.concatenate(
                [acc[0], acc[1, :N_NODE - SPLIT]], axis=0))
        return outs

    accA = gat_edge(hs_drug, av_drug, ad_drug, ae_drug, sa_g, da_g,
                    E_ATOM, 20, 100)[0]
    accB = gat_edge(hs_int_a, av_int_a, ad_int_atom, None, mp_g, ms_g,
                    E_M2P, 20, 100)[0]

    biasD = _pad_cols(pD['bias'].reshape(1, -1), 128)
    biasI = _pad_cols(pI['bias'].reshape(1, -1), 128)
    xcat_a = _finalize_concat(accA, accB, biasD, biasI)
    stats_a = _gln_stats(xcat_a, batch_a)
    atom_h = _gln_norm_elu(xcat_a, batch_a, stats_a.T,
                           params['drug_norm_w'].reshape(1, -1),
                           params['drug_norm_b'].reshape(1, -1))

    W_ah = jnp.concatenate([
        _pad_cols(pL['W_src'], 256),
        _fold_att(pL['W_src'], pL['att_src']),
        _fold_att(pL['W_dst'], pL['att_dst']),
        _pad_cols(pI['W_src'], 128),
        _fold_att(pI['W_src'], pI['att_src']),
    ], axis=1)
    hcat = _mm(atom_h, W_ah)
    hs_pool = hcat[:, :256].at[:, 200:205].set(1.0)
    av_pool = hcat[:, 256:261].reshape(-1)
    ad_pool = hcat[:, 261:266].reshape(-1)
    hs_int_h = ones_cols(hcat[:, 266:394])
    av_int_h = hcat[:, 394:399].reshape(-1)

    accC = gat_edge(hs_prot, av_prot, ad_prot, ae_prot, sp_g, dp_g,
                    E_AA, 20, 100)[0]
    accD = gat_edge(hs_int_h, av_int_h, ad_int_aa, None, ms_g, mp_g,
                    E_M2P, 20, 100)[0]
    accP = gat_edge(hs_pool, av_pool, ad_pool, ae_pool, sa_g, da_g,
                    E_ATOM, 40, 200)

    biasP = _pad_cols(pP['bias'].reshape(1, -1), 128)
    xcat_p = _finalize_concat(accC, accD, biasP, biasI)
    stats_p = _gln_stats(xcat_p, batch_p)
    aa_h = _gln_norm_elu(xcat_p, batch_p, stats_p.T,
                         params['prot_norm_w'].reshape(1, -1),
                         params['prot_norm_b'].reshape(1, -1))

    W_sag = jnp.concatenate([
        _pad_cols(params['sag_W_rel'], 4), _pad_cols(params['sag_W_root'], 4),
    ], axis=1)
    tr = _mm(aa_h, W_sag)
    t = tr[:, 0]
    root_b = tr[:, 4:5] + params['sag_b_rel'][0]

    rel = _make_sc_seg(E_AA)(t, sp_g, dp_g).reshape(32, 10240)
    rel3 = rel[:, :N_NODE, None]

    e_s, dsum = _score_exp(rel3, root_b, batch_p)
    aa_out, prot_g = _aa_final(e_s, dsum.reshape(128, 1), aa_h, aa_x, batch_p)

    biasL = _pad_cols(pL['bias'].reshape(1, -1), 256)
    atom_out, drug_g = _atom_final(accP[0], accP[1], biasL, atom_x, batch_a)

    return (atom_out, aa_out, drug_g, prot_g)
